# baseline (speedup 1.0000x reference)
import functools

import jax
import jax.numpy as jnp
from jax import lax
from jax.experimental import pallas as pl
from jax.experimental.pallas import tpu as pltpu
from jax.experimental.pallas import tpu_sc as plsc

NC, NS, LANES = 2, 16, 16
NW = NC * NS
NP = 10240
RPT = NP // NW
TROWS = RPT + 1
H = 256
Q = 64
FLUSH = 8192
LBUF = FLUSH + 96
ECAP = 19 * FLUSH + LBUF + 64

_mesh = plsc.VectorSubcoreMesh(core_axis_name="c", subcore_axis_name="s")
_SC_PARAMS = pltpu.CompilerParams(needs_layout_passes=False)
_SC_PARAMS_NT = pltpu.CompilerParams(needs_layout_passes=False,
                                     use_tc_tiling_on_sc=False)


def _wid():
    return lax.axis_index("s") * NC + lax.axis_index("c")


def _build_body(ei, srcl, eidl, dstl, cnts, stage_s, stage_d,
                bufs, bufe, bufd, cw):
    E = ei.shape[0] // 2
    BLK = 2000
    nblk = E // BLK
    wid = _wid()
    n0 = wid * RPT
    n1 = n0 + RPT
    lb = wid * ECAP
    iota = lax.broadcasted_iota(jnp.int32, (LANES,), 0)

    def blk_body(blk, carry):
        cnt, wr = carry
        pltpu.sync_copy(ei.at[pl.ds(blk * BLK, BLK)], stage_s)
        pltpu.sync_copy(ei.at[pl.ds(E + blk * BLK, BLK)], stage_d)

        def chunk_body(ci, carry2):
            cnt, wr = carry2
            d = stage_d[pl.ds(ci * LANES, LANES)]
            s = stage_s[pl.ds(ci * LANES, LANES)]
            e = blk * BLK + ci * LANES + iota
            m = (d >= n0) & (d < n1)
            mi = m.astype(jnp.int32)
            pos = cnt + plsc.cumsum(mi) - mi
            plsc.store_scatter(bufd, [pos], d - n0, mask=m)
            plsc.store_scatter(bufs, [pos], s, mask=m)
            plsc.store_scatter(bufe, [pos], e, mask=m)
            cnt = cnt + jnp.sum(mi)

            def do_flush(c_w):
                c, w = c_w
                w = pl.multiple_of(w, FLUSH)
                pltpu.sync_copy(bufd.at[pl.ds(0, FLUSH)],
                                dstl.at[pl.ds(lb + w, FLUSH)])
                pltpu.sync_copy(bufs.at[pl.ds(0, FLUSH)],
                                srcl.at[pl.ds(lb + w, FLUSH)])
                pltpu.sync_copy(bufe.at[pl.ds(0, FLUSH)],
                                eidl.at[pl.ds(lb + w, FLUSH)])
                td = bufd[pl.ds(FLUSH, LANES)]
                ts = bufs[pl.ds(FLUSH, LANES)]
                te = bufe[pl.ds(FLUSH, LANES)]
                bufd[pl.ds(0, LANES)] = td
                bufs[pl.ds(0, LANES)] = ts
                bufe[pl.ds(0, LANES)] = te
                return (c - FLUSH, w + FLUSH)

            return lax.cond(cnt >= FLUSH, do_flush, lambda c_w: c_w,
                            (cnt, wr))

        return lax.fori_loop(0, BLK // LANES, chunk_body, (cnt, wr))

    cnt, wr = lax.fori_loop(0, nblk, blk_body, (0, 0))
    wr = pl.multiple_of(wr, FLUSH)

    sent_d = jnp.full((LANES,), RPT, jnp.int32)
    zer = jnp.zeros((LANES,), jnp.int32)
    for i in range(4):
        bufd[pl.ds(cnt + i * LANES, LANES)] = sent_d
        bufs[pl.ds(cnt + i * LANES, LANES)] = zer
        bufe[pl.ds(cnt + i * LANES, LANES)] = zer
    total = wr + cnt
    padded = ((total + 63) // 64) * 64
    pltpu.sync_copy(bufd.at[pl.ds(0, LBUF)], dstl.at[pl.ds(lb + wr, LBUF)])
    pltpu.sync_copy(bufs.at[pl.ds(0, LBUF)], srcl.at[pl.ds(lb + wr, LBUF)])
    pltpu.sync_copy(bufe.at[pl.ds(0, LBUF)], eidl.at[pl.ds(lb + wr, LBUF)])
    cw[pl.ds(0, LANES)] = jnp.full((LANES,), padded, jnp.int32)
    pltpu.sync_copy(cw, cnts.at[pl.ds(wid * LANES, LANES)])


def _build_lists(ei):
    E = ei.shape[1]
    out_type = [
        jax.ShapeDtypeStruct((NW * ECAP,), jnp.int32),
        jax.ShapeDtypeStruct((NW * ECAP,), jnp.int32),
        jax.ShapeDtypeStruct((NW * ECAP,), jnp.int32),
        jax.ShapeDtypeStruct((NW * LANES,), jnp.int32),
    ]
    scratch = [
        pltpu.VMEM((2000,), jnp.int32),
        pltpu.VMEM((2000,), jnp.int32),
        pltpu.VMEM((LBUF,), jnp.int32),
        pltpu.VMEM((LBUF,), jnp.int32),
        pltpu.VMEM((LBUF,), jnp.int32),
        pltpu.VMEM((LANES,), jnp.int32),
    ]
    return pl.kernel(_build_body, out_type=out_type, mesh=_mesh,
                     scratch_types=scratch,
                     compiler_params=_SC_PARAMS)(ei.reshape(2 * E))


def _layer_body(hv4, ea4, srcl, eidl, dstl, cnts, tvec, aggr,
                tab_m, tab_d, tab_n, srcb, eidb, dstb, idxb,
                hq, eq, outq, cbuf, tbuf):
    wid = _wid()
    n0 = wid * RPT
    lb = wid * ECAP
    pltpu.sync_copy(cnts, cbuf)
    pltpu.sync_copy(tvec, tbuf)
    count = cbuf[pl.ds(wid * LANES, LANES)][0]
    nb = count // 64
    neg = jnp.full((LANES,), -3e38, jnp.float32)
    zf = jnp.zeros((LANES,), jnp.float32)

    for q in range(4):
        def init_body(i, _):
            tab_m[pl.ds(i * LANES, LANES)] = neg
            tab_d[pl.ds(i * LANES, LANES)] = zf
            tab_n[pl.ds(i * LANES, LANES)] = zf
            return 0
        lax.fori_loop(0, (TROWS * Q) // LANES, init_body, 0)

        def batch_body(b, _):
            off = b * 64
            pltpu.sync_copy(srcl.at[pl.ds(lb + off, 64)], srcb)
            pltpu.sync_copy(eidl.at[pl.ds(lb + off, 64)], eidb)
            pltpu.sync_copy(dstl.at[pl.ds(lb + off, 64)], dstb)
            for c in range(4):
                v = srcb[pl.ds(c * LANES, LANES)]
                idxb[pl.ds(c * LANES, LANES)] = v * 4 + q
            pltpu.sync_copy(hv4.at[idxb], hq)
            for c in range(4):
                v = eidb[pl.ds(c * LANES, LANES)]
                idxb[pl.ds(c * LANES, LANES)] = v * 4 + q
            pltpu.sync_copy(ea4.at[idxb], eq)

            def grp_body(g, _):
                dvec = dstb[pl.ds(g * LANES, LANES)]
                tv = tbuf[pl.ds(0, LANES)]
                for jj in range(LANES):
                    j = g * LANES + jj
                    base = dvec[jj] * Q
                    for k in range(4):
                        hh = hq[j, pl.ds(k * LANES, LANES)]
                        ee = eq[j, pl.ds(k * LANES, LANES)]
                        msg = jnp.maximum(hh + ee, 0.0) + 1e-7
                        s = msg * tv
                        mo = tab_m[pl.ds(base + k * LANES, LANES)]
                        mn = jnp.maximum(mo, s)
                        e1 = jnp.exp(s - mn)
                        sc = jnp.exp(mo - mn)
                        dd = tab_d[pl.ds(base + k * LANES, LANES)]
                        nn = tab_n[pl.ds(base + k * LANES, LANES)]
                        tab_d[pl.ds(base + k * LANES, LANES)] = dd * sc + e1
                        tab_n[pl.ds(base + k * LANES, LANES)] = (nn * sc
                                                                 + msg * e1)
                        tab_m[pl.ds(base + k * LANES, LANES)] = mn
                return 0
            lax.fori_loop(0, 4, grp_body, 0)
            return 0
        lax.fori_loop(0, nb, batch_body, 0)

        def aggr_body(i, _):
            r = i // (Q // LANES)
            col = (i % (Q // LANES)) * LANES
            dd = tab_d[pl.ds(i * LANES, LANES)]
            nn = tab_n[pl.ds(i * LANES, LANES)]
            outq[r, pl.ds((q % 2) * Q + col, LANES)] = nn / (dd + 1e-16)
            return 0
        lax.fori_loop(0, (RPT * Q) // LANES, aggr_body, 0)
        if q % 2 == 1:
            pltpu.sync_copy(
                outq, aggr.at[pl.ds(n0, RPT), pl.ds((q // 2) * 2 * Q, 2 * Q)])


def _sc_layer(hv, ea, srcl, eidl, dstl, cnts, t):
    E = ea.shape[0]
    hv4 = hv.reshape(4 * NP, Q)
    ea4 = ea.reshape(4 * E, Q)
    tvec = jnp.full((LANES,), t, jnp.float32)
    scratch = [
        pltpu.VMEM((TROWS * Q,), jnp.float32),
        pltpu.VMEM((TROWS * Q,), jnp.float32),
        pltpu.VMEM((TROWS * Q,), jnp.float32),
        pltpu.VMEM((64,), jnp.int32),
        pltpu.VMEM((64,), jnp.int32),
        pltpu.VMEM((64,), jnp.int32),
        pltpu.VMEM((64,), jnp.int32),
        pltpu.VMEM((64, Q), jnp.float32),
        pltpu.VMEM((64, Q), jnp.float32),
        pltpu.VMEM((RPT, 2 * Q), jnp.float32),
        pltpu.VMEM((NW * LANES,), jnp.int32),
        pltpu.VMEM((LANES,), jnp.float32),
    ]
    return pl.kernel(_layer_body,
                     out_type=jax.ShapeDtypeStruct((NP, H), jnp.float32),
                     mesh=_mesh, scratch_types=scratch,
                     compiler_params=_SC_PARAMS_NT)(
                         hv4, ea4, srcl, eidl, dstl, cnts, tvec)


def _final_body(a_t, b_t, ei, fp, out, srcb, dstb, ha, hb, fbuf, outb):
    Ep = ei.shape[0] // 2
    ept = Ep // NW
    wid = _wid()
    e0 = wid * ept
    pltpu.sync_copy(fp, fbuf)

    def batch_body(b, _):
        off = b * 64
        pltpu.sync_copy(ei.at[pl.ds(e0 + off, 64)], srcb)
        pltpu.sync_copy(ei.at[pl.ds(Ep + e0 + off, 64)], dstb)
        pltpu.sync_copy(a_t.at[srcb], ha)
        pltpu.sync_copy(b_t.at[dstb], hb)
        iota = lax.broadcasted_iota(jnp.int32, (LANES,), 0)

        def grp_body(g, _):
            vec = jnp.zeros((LANES,), jnp.float32)
            for jj in range(8):
                j = g * 8 + jj
                acc0 = jnp.zeros((LANES,), jnp.float32)
                acc1 = jnp.zeros((LANES,), jnp.float32)
                for k in range(H // LANES):
                    z = jnp.maximum(ha[j, pl.ds(k * LANES, LANES)]
                                    + hb[j, pl.ds(k * LANES, LANES)], 0.0)
                    acc0 = acc0 + z * fbuf[1, pl.ds(k * LANES, LANES)]
                    acc1 = acc1 + z * fbuf[2, pl.ds(k * LANES, LANES)]
                b2v = fbuf[3, pl.ds(0, LANES)]
                o0 = jnp.sum(acc0) + b2v[0]
                o1 = jnp.sum(acc1) + b2v[1]
                vec = (vec + jnp.where(iota == 2 * jj, o0, 0.0)
                       + jnp.where(iota == 2 * jj + 1, o1, 0.0))
            outb[pl.ds((off + g * 8) * 2, LANES)] = vec
            return 0
        lax.fori_loop(0, 8, grp_body, 0)
        return 0
    lax.fori_loop(0, ept // 64, batch_body, 0)
    pltpu.sync_copy(outb, out.at[pl.ds(e0 * 2, ept * 2)])


def _sc_final(a_t, b_t, ei_p, fp):
    Ep = ei_p.shape[1]
    ept = Ep // NW
    scratch = [
        pltpu.VMEM((64,), jnp.int32),
        pltpu.VMEM((64,), jnp.int32),
        pltpu.VMEM((64, H), jnp.float32),
        pltpu.VMEM((64, H), jnp.float32),
        pltpu.VMEM((4, H), jnp.float32),
        pltpu.VMEM((ept * 2,), jnp.float32),
    ]
    out = pl.kernel(_final_body,
                    out_type=jax.ShapeDtypeStruct((Ep * 2,), jnp.float32),
                    mesh=_mesh, scratch_types=scratch,
                    compiler_params=_SC_PARAMS)(
                        a_t, b_t, ei_p.reshape(2 * Ep), fp)
    return out.reshape(Ep, 2)


def _enc_h_body(x_ref, w_ref, b_ref, o_ref):
    o_ref[...] = (jnp.dot(x_ref[...], w_ref[...],
                          preferred_element_type=jnp.float32) + b_ref[...])


def _enc_h(x_p, w, b):
    BN = 1280
    return pl.pallas_call(
        _enc_h_body,
        grid=(NP // BN,),
        in_specs=[
            pl.BlockSpec((BN, x_p.shape[1]), lambda i: (i, 0)),
            pl.BlockSpec((x_p.shape[1], H), lambda i: (0, 0)),
            pl.BlockSpec((1, H), lambda i: (0, 0)),
        ],
        out_specs=pl.BlockSpec((BN, H), lambda i: (i, 0)),
        out_shape=jax.ShapeDtypeStruct((NP, H), jnp.float32),
    )(x_p, w, b.reshape(1, H))


def _enc_ea(ea_attr, w, b):
    E, DE = ea_attr.shape
    BE = 2000
    return pl.pallas_call(
        _enc_h_body,
        grid=(E // BE,),
        in_specs=[
            pl.BlockSpec((BE, DE), lambda i: (i, 0)),
            pl.BlockSpec((DE, H), lambda i: (0, 0)),
            pl.BlockSpec((1, H), lambda i: (0, 0)),
        ],
        out_specs=pl.BlockSpec((BE, H), lambda i: (i, 0)),
        out_shape=jax.ShapeDtypeStruct((E, H), jnp.float32),
    )(ea_attr, w, b.reshape(1, H))


def _layer_dense_body(aggr_ref, hv_ref, hp_ref, w1_ref, b1_ref, g1_ref,
                      be1_ref, w2_ref, b2_ref, ng_ref, nb_ref,
                      hn_ref, r_ref):
    o = aggr_ref[...] + hv_ref[...]
    u = jnp.dot(o, w1_ref[...], preferred_element_type=jnp.float32) + b1_ref[...]
    m = jnp.mean(u, axis=-1, keepdims=True)
    v = jnp.mean((u - m) ** 2, axis=-1, keepdims=True)
    u = (u - m) * lax.rsqrt(v + 1e-5) * g1_ref[...] + be1_ref[...]
    u = jnp.maximum(u, 0.0)
    y = jnp.dot(u, w2_ref[...], preferred_element_type=jnp.float32) + b2_ref[...]
    hn = hp_ref[...] + y
    hn_ref[...] = hn
    m2 = jnp.mean(hn, axis=-1, keepdims=True)
    v2 = jnp.mean((hn - m2) ** 2, axis=-1, keepdims=True)
    r_ref[...] = jnp.maximum(
        (hn - m2) * lax.rsqrt(v2 + 1e-5) * ng_ref[...] + nb_ref[...], 0.0)


def _layer_dense(aggr, hv, hp, cp, ng, nb):
    BN = 1280
    H2 = 2 * H
    row = lambda a, n: a.reshape(1, n)
    return pl.pallas_call(
        _layer_dense_body,
        grid=(NP // BN,),
        in_specs=[
            pl.BlockSpec((BN, H), lambda i: (i, 0)),
            pl.BlockSpec((BN, H), lambda i: (i, 0)),
            pl.BlockSpec((BN, H), lambda i: (i, 0)),
            pl.BlockSpec((H, H2), lambda i: (0, 0)),
            pl.BlockSpec((1, H2), lambda i: (0, 0)),
            pl.BlockSpec((1, H2), lambda i: (0, 0)),
            pl.BlockSpec((1, H2), lambda i: (0, 0)),
            pl.BlockSpec((H2, H), lambda i: (0, 0)),
            pl.BlockSpec((1, H), lambda i: (0, 0)),
            pl.BlockSpec((1, H), lambda i: (0, 0)),
            pl.BlockSpec((1, H), lambda i: (0, 0)),
        ],
        out_specs=[
            pl.BlockSpec((BN, H), lambda i: (i, 0)),
            pl.BlockSpec((BN, H), lambda i: (i, 0)),
        ],
        out_shape=[
            jax.ShapeDtypeStruct((NP, H), jnp.float32),
            jax.ShapeDtypeStruct((NP, H), jnp.float32),
        ],
    )(aggr, hv, hp, cp["W1"], row(cp["b1"], H2), row(cp["g1"], H2),
      row(cp["be1"], H2), cp["W2"], row(cp["b2"], H), row(ng, H), row(nb, H))


def _ab_body(hf_ref, w1a_ref, w1b_ref, b1_ref, a_ref, b_ref):
    hf = hf_ref[...]
    a_ref[...] = (jnp.dot(hf, w1a_ref[...],
                          preferred_element_type=jnp.float32) + b1_ref[...])
    b_ref[...] = jnp.dot(hf, w1b_ref[...], preferred_element_type=jnp.float32)


def _ab(hf, w1a, w1b, b1):
    BN = 1280
    return pl.pallas_call(
        _ab_body,
        grid=(NP // BN,),
        in_specs=[
            pl.BlockSpec((BN, H), lambda i: (i, 0)),
            pl.BlockSpec((H, H), lambda i: (0, 0)),
            pl.BlockSpec((H, H), lambda i: (0, 0)),
            pl.BlockSpec((1, H), lambda i: (0, 0)),
        ],
        out_specs=[
            pl.BlockSpec((BN, H), lambda i: (i, 0)),
            pl.BlockSpec((BN, H), lambda i: (i, 0)),
        ],
        out_shape=[
            jax.ShapeDtypeStruct((NP, H), jnp.float32),
            jax.ShapeDtypeStruct((NP, H), jnp.float32),
        ],
    )(hf, w1a, w1b, b1.reshape(1, H))


def kernel(x, edge_index, edge_attr, params):
    N = x.shape[0]
    E = edge_index.shape[1]
    layers = params["layers"]

    srcl, eidl, dstl, cnts = _build_lists(edge_index)

    x_p = jnp.pad(x, ((0, NP - N), (0, 0)))
    h0 = _enc_h(x_p, params["node_enc"]["W"], params["node_enc"]["b"])
    ea = _enc_ea(edge_attr, params["edge_enc"]["W"], params["edge_enc"]["b"])

    zeros_n = jnp.zeros((NP, H), jnp.float32)
    hv = h0
    hp = zeros_n
    for i in range(3):
        cp = layers[i]["conv"]
        nxt = layers[i + 1] if i < 2 else layers[0]
        aggr = _sc_layer(hv, ea, srcl, eidl, dstl, cnts, cp["t"])
        hp, hv = _layer_dense(aggr, hv, hp, cp,
                              nxt["norm_g"], nxt["norm_b"])
    hf = hv

    m = params["mlp"]
    a_t, b_t = _ab(hf, m["W1"][:H, :], m["W1"][H:, :], m["b1"])

    EP = ((E + 64 * NW - 1) // (64 * NW)) * (64 * NW)
    ei_p = jnp.pad(edge_index, ((0, 0), (0, EP - E)))
    fp = jnp.zeros((4, H), jnp.float32)
    fp = fp.at[1].set(m["W2"][:, 0])
    fp = fp.at[2].set(m["W2"][:, 1])
    fp = fp.at[3, 0:2].set(m["b2"])
    out = _sc_final(a_t, b_t, ei_p, fp)
    return out[:E]


# speedup vs baseline: 1.1908x; 1.1908x over previous
import functools

import jax
import jax.numpy as jnp
from jax import lax
from jax.experimental import pallas as pl
from jax.experimental.pallas import tpu as pltpu
from jax.experimental.pallas import tpu_sc as plsc

NC, NS, LANES = 2, 16, 16
NW = NC * NS
NP = 10240
RPT = NP // NW
TROWS = RPT + 1
H = 256
Q = 64
BE = 128
FLUSH = 8192
LBUF = FLUSH + 272
ECAP = 19 * FLUSH + LBUF + 64

_mesh = plsc.VectorSubcoreMesh(core_axis_name="c", subcore_axis_name="s")
_SC_PARAMS = pltpu.CompilerParams(needs_layout_passes=False)
_SC_PARAMS_NT = pltpu.CompilerParams(needs_layout_passes=False,
                                     use_tc_tiling_on_sc=False)


def _wid():
    return lax.axis_index("s") * NC + lax.axis_index("c")


def _build_body(ei, srcl, eidl, dstl, cnts, stage_s, stage_d,
                bufs, bufe, bufd, cw):
    E = ei.shape[0] // 2
    BLK = 2000
    nblk = E // BLK
    wid = _wid()
    n0 = wid * RPT
    n1 = n0 + RPT
    lb = wid * ECAP
    iota = lax.broadcasted_iota(jnp.int32, (LANES,), 0)

    def blk_body(blk, carry):
        cnt, wr = carry
        pltpu.sync_copy(ei.at[pl.ds(blk * BLK, BLK)], stage_s)
        pltpu.sync_copy(ei.at[pl.ds(E + blk * BLK, BLK)], stage_d)

        def chunk_body(ci, carry2):
            cnt, wr = carry2
            d = stage_d[pl.ds(ci * LANES, LANES)]
            s = stage_s[pl.ds(ci * LANES, LANES)]
            e = blk * BLK + ci * LANES + iota
            m = (d >= n0) & (d < n1)
            mi = m.astype(jnp.int32)
            pos = cnt + plsc.cumsum(mi) - mi
            plsc.store_scatter(bufd, [pos], d - n0, mask=m)
            plsc.store_scatter(bufs, [pos], s, mask=m)
            plsc.store_scatter(bufe, [pos], e, mask=m)
            cnt = cnt + jnp.sum(mi)

            def do_flush(c_w):
                c, w = c_w
                w = pl.multiple_of(w, FLUSH)
                pltpu.sync_copy(bufd.at[pl.ds(0, FLUSH)],
                                dstl.at[pl.ds(lb + w, FLUSH)])
                pltpu.sync_copy(bufs.at[pl.ds(0, FLUSH)],
                                srcl.at[pl.ds(lb + w, FLUSH)])
                pltpu.sync_copy(bufe.at[pl.ds(0, FLUSH)],
                                eidl.at[pl.ds(lb + w, FLUSH)])
                td = bufd[pl.ds(FLUSH, LANES)]
                ts = bufs[pl.ds(FLUSH, LANES)]
                te = bufe[pl.ds(FLUSH, LANES)]
                bufd[pl.ds(0, LANES)] = td
                bufs[pl.ds(0, LANES)] = ts
                bufe[pl.ds(0, LANES)] = te
                return (c - FLUSH, w + FLUSH)

            return lax.cond(cnt >= FLUSH, do_flush, lambda c_w: c_w,
                            (cnt, wr))

        return lax.fori_loop(0, BLK // LANES, chunk_body, (cnt, wr))

    cnt, wr = lax.fori_loop(0, nblk, blk_body, (0, 0))
    wr = pl.multiple_of(wr, FLUSH)

    sent_d = jnp.full((LANES,), RPT, jnp.int32)
    zer = jnp.zeros((LANES,), jnp.int32)
    for i in range(16):
        bufd[pl.ds(cnt + i * LANES, LANES)] = sent_d
        bufs[pl.ds(cnt + i * LANES, LANES)] = zer
        bufe[pl.ds(cnt + i * LANES, LANES)] = zer
    total = wr + cnt
    padded = ((total + 255) // 256) * 256
    pltpu.sync_copy(bufd.at[pl.ds(0, LBUF)], dstl.at[pl.ds(lb + wr, LBUF)])
    pltpu.sync_copy(bufs.at[pl.ds(0, LBUF)], srcl.at[pl.ds(lb + wr, LBUF)])
    pltpu.sync_copy(bufe.at[pl.ds(0, LBUF)], eidl.at[pl.ds(lb + wr, LBUF)])
    cw[pl.ds(0, LANES)] = jnp.full((LANES,), padded, jnp.int32)
    pltpu.sync_copy(cw, cnts.at[pl.ds(wid * LANES, LANES)])


def _build_lists(ei):
    E = ei.shape[1]
    out_type = [
        jax.ShapeDtypeStruct((NW * ECAP,), jnp.int32),
        jax.ShapeDtypeStruct((NW * ECAP,), jnp.int32),
        jax.ShapeDtypeStruct((NW * ECAP,), jnp.int32),
        jax.ShapeDtypeStruct((NW * LANES,), jnp.int32),
    ]
    scratch = [
        pltpu.VMEM((2000,), jnp.int32),
        pltpu.VMEM((2000,), jnp.int32),
        pltpu.VMEM((LBUF,), jnp.int32),
        pltpu.VMEM((LBUF,), jnp.int32),
        pltpu.VMEM((LBUF,), jnp.int32),
        pltpu.VMEM((LANES,), jnp.int32),
    ]
    return pl.kernel(_build_body, out_type=out_type, mesh=_mesh,
                     scratch_types=scratch,
                     compiler_params=_SC_PARAMS)(ei.reshape(2 * E))


def _layer_body(hv4, ea4, srcl, eidl, dstl, cnts, tvec, aggr,
                tab_m, tab_d, tab_n,
                lbs, lbe, lbd,
                dstb0, dstb1, idxh0, idxh1, idxe0, idxe1,
                hq0, hq1, eq0, eq1, cbuf, tbuf,
                semh0, semh1, seme0, seme1):
    wid = _wid()
    n0 = wid * RPT
    lb = wid * ECAP
    pltpu.sync_copy(cnts, cbuf)
    pltpu.sync_copy(tvec, tbuf)
    count = cbuf[pl.ds(wid * LANES, LANES)][0]
    nb = count // BE
    neg = jnp.full((LANES,), -3e38, jnp.float32)
    zf = jnp.zeros((LANES,), jnp.float32)
    slots = ((dstb0, idxh0, idxe0, hq0, eq0, semh0, seme0),
             (dstb1, idxh1, idxe1, hq1, eq1, semh1, seme1))

    def quarter_body(q, _):
        def init_body(i, _):
            tab_m[i, pl.ds(0, LANES)] = neg
            tab_m[i, pl.ds(LANES, LANES)] = neg
            tab_m[i, pl.ds(2 * LANES, LANES)] = neg
            tab_m[i, pl.ds(3 * LANES, LANES)] = neg
            for kk in range(4):
                tab_d[i, pl.ds(kk * LANES, LANES)] = zf
                tab_n[i, pl.ds(kk * LANES, LANES)] = zf
            return 0
        lax.fori_loop(0, TROWS, init_body, 0)

        def load_block(j):
            boff = lb + j * (8 * BE)
            pltpu.sync_copy(srcl.at[pl.ds(boff, 8 * BE)], lbs)
            pltpu.sync_copy(eidl.at[pl.ds(boff, 8 * BE)], lbe)
            pltpu.sync_copy(dstl.at[pl.ds(boff, 8 * BE)], lbd)

        def issue(bn, slot):
            dstb, idxh, idxe, hq, eq, semh, seme = slot
            o = (bn % 8) * BE
            for c in range(BE // LANES):
                v = lbs[pl.ds(o + c * LANES, LANES)]
                idxh[pl.ds(c * LANES, LANES)] = v * 4 + q
                w = lbe[pl.ds(o + c * LANES, LANES)]
                idxe[pl.ds(c * LANES, LANES)] = w * 4 + q
                dstb[pl.ds(c * LANES, LANES)] = lbd[pl.ds(o + c * LANES,
                                                          LANES)]
            pltpu.async_copy(hv4.at[idxh], hq, semh)
            pltpu.async_copy(ea4.at[idxe], eq, seme)

        def compute(slot):
            dstb, idxh, idxe, hq, eq, semh, seme = slot
            pltpu.make_async_copy(hv4.at[idxh], hq, semh).wait()
            pltpu.make_async_copy(ea4.at[idxe], eq, seme).wait()
            tv = tbuf[pl.ds(0, LANES)]

            def grp_body(g, _):
                dvec = dstb[pl.ds(g * LANES, LANES)]
                for jj in range(LANES):
                    j = g * LANES + jj
                    r = dvec[jj]
                    for k in range(4):
                        hh = hq[j, pl.ds(k * LANES, LANES)]
                        ee = eq[j, pl.ds(k * LANES, LANES)]
                        msg = jnp.maximum(hh + ee, 0.0) + 1e-7
                        s = msg * tv
                        mo = tab_m[r, pl.ds(k * LANES, LANES)]
                        mn = jnp.maximum(mo, s)
                        e1 = jnp.exp(s - mn)
                        sc = jnp.exp(mo - mn)
                        dd = tab_d[r, pl.ds(k * LANES, LANES)]
                        nn = tab_n[r, pl.ds(k * LANES, LANES)]
                        tab_d[r, pl.ds(k * LANES, LANES)] = dd * sc + e1
                        tab_n[r, pl.ds(k * LANES, LANES)] = nn * sc + msg * e1
                        tab_m[r, pl.ds(k * LANES, LANES)] = mn
                return 0
            lax.fori_loop(0, BE // LANES, grp_body, 0)

        @pl.when(nb > 0)
        def _():
            load_block(0)
            issue(0, slots[0])

        @pl.when(nb > 1)
        def _():
            issue(1, slots[1])

        def pair_body(i, _):
            compute(slots[0])
            bn = 2 * i + 2

            @pl.when(bn < nb)
            def _():
                @pl.when(bn % 8 == 0)
                def _():
                    load_block(bn // 8)
                issue(bn, slots[0])

            compute(slots[1])

            @pl.when(bn + 1 < nb)
            def _():
                issue(bn + 1, slots[1])
            return 0
        lax.fori_loop(0, nb // 2, pair_body, 0)

        def aggr_body(i, _):
            for kk in range(4):
                dd = tab_d[i, pl.ds(kk * LANES, LANES)]
                nn = tab_n[i, pl.ds(kk * LANES, LANES)]
                tab_n[i, pl.ds(kk * LANES, LANES)] = nn / (dd + 1e-16)
            return 0
        lax.fori_loop(0, RPT, aggr_body, 0)
        pltpu.sync_copy(tab_n.at[pl.ds(0, RPT), :],
                        aggr.at[pl.ds(n0, RPT), pl.ds(q * Q, Q)])
        return 0

    lax.fori_loop(0, 4, quarter_body, 0)


def _sc_layer(hv, ea, srcl, eidl, dstl, cnts, t):
    E = ea.shape[0]
    hv4 = hv.reshape(4 * NP, Q)
    ea4 = ea.reshape(4 * E, Q)
    tvec = jnp.full((LANES,), t, jnp.float32)
    scratch = [
        pltpu.VMEM((TROWS, Q), jnp.float32),
        pltpu.VMEM((TROWS, Q), jnp.float32),
        pltpu.VMEM((TROWS, Q), jnp.float32),
        pltpu.VMEM((8 * BE,), jnp.int32),
        pltpu.VMEM((8 * BE,), jnp.int32),
        pltpu.VMEM((8 * BE,), jnp.int32),
        pltpu.VMEM((BE,), jnp.int32),
        pltpu.VMEM((BE,), jnp.int32),
        pltpu.VMEM((BE,), jnp.int32),
        pltpu.VMEM((BE,), jnp.int32),
        pltpu.VMEM((BE,), jnp.int32),
        pltpu.VMEM((BE,), jnp.int32),
        pltpu.VMEM((BE, Q), jnp.float32),
        pltpu.VMEM((BE, Q), jnp.float32),
        pltpu.VMEM((BE, Q), jnp.float32),
        pltpu.VMEM((BE, Q), jnp.float32),
        pltpu.VMEM((NW * LANES,), jnp.int32),
        pltpu.VMEM((LANES,), jnp.float32),
        pltpu.SemaphoreType.DMA,
        pltpu.SemaphoreType.DMA,
        pltpu.SemaphoreType.DMA,
        pltpu.SemaphoreType.DMA,
    ]
    return pl.kernel(_layer_body,
                     out_type=jax.ShapeDtypeStruct((NP, H), jnp.float32),
                     mesh=_mesh, scratch_types=scratch,
                     compiler_params=_SC_PARAMS_NT)(
                         hv4, ea4, srcl, eidl, dstl, cnts, tvec)


def _final_body(a_t, b_t, ei, fp, out, srcb, dstb, ha, hb, fbuf, outb):
    Ep = ei.shape[0] // 2
    ept = Ep // NW
    wid = _wid()
    e0 = wid * ept
    pltpu.sync_copy(fp, fbuf)

    def batch_body(b, _):
        off = b * 64
        pltpu.sync_copy(ei.at[pl.ds(e0 + off, 64)], srcb)
        pltpu.sync_copy(ei.at[pl.ds(Ep + e0 + off, 64)], dstb)
        pltpu.sync_copy(a_t.at[srcb], ha)
        pltpu.sync_copy(b_t.at[dstb], hb)
        iota = lax.broadcasted_iota(jnp.int32, (LANES,), 0)

        def grp_body(g, _):
            vec = jnp.zeros((LANES,), jnp.float32)
            for jj in range(8):
                j = g * 8 + jj
                acc0 = jnp.zeros((LANES,), jnp.float32)
                acc1 = jnp.zeros((LANES,), jnp.float32)
                for k in range(H // LANES):
                    z = jnp.maximum(ha[j, pl.ds(k * LANES, LANES)]
                                    + hb[j, pl.ds(k * LANES, LANES)], 0.0)
                    acc0 = acc0 + z * fbuf[1, pl.ds(k * LANES, LANES)]
                    acc1 = acc1 + z * fbuf[2, pl.ds(k * LANES, LANES)]
                b2v = fbuf[3, pl.ds(0, LANES)]
                o0 = jnp.sum(acc0) + b2v[0]
                o1 = jnp.sum(acc1) + b2v[1]
                vec = (vec + jnp.where(iota == 2 * jj, o0, 0.0)
                       + jnp.where(iota == 2 * jj + 1, o1, 0.0))
            outb[pl.ds((off + g * 8) * 2, LANES)] = vec
            return 0
        lax.fori_loop(0, 8, grp_body, 0)
        return 0
    lax.fori_loop(0, ept // 64, batch_body, 0)
    pltpu.sync_copy(outb, out.at[pl.ds(e0 * 2, ept * 2)])


def _sc_final(a_t, b_t, ei_p, fp):
    Ep = ei_p.shape[1]
    ept = Ep // NW
    scratch = [
        pltpu.VMEM((64,), jnp.int32),
        pltpu.VMEM((64,), jnp.int32),
        pltpu.VMEM((64, H), jnp.float32),
        pltpu.VMEM((64, H), jnp.float32),
        pltpu.VMEM((4, H), jnp.float32),
        pltpu.VMEM((ept * 2,), jnp.float32),
    ]
    out = pl.kernel(_final_body,
                    out_type=jax.ShapeDtypeStruct((Ep * 2,), jnp.float32),
                    mesh=_mesh, scratch_types=scratch,
                    compiler_params=_SC_PARAMS)(
                        a_t, b_t, ei_p.reshape(2 * Ep), fp)
    return out.reshape(Ep, 2)


def _enc_h_body(x_ref, w_ref, b_ref, o_ref):
    o_ref[...] = (jnp.dot(x_ref[...], w_ref[...],
                          preferred_element_type=jnp.float32) + b_ref[...])


def _enc_h(x_p, w, b):
    BN = 1280
    return pl.pallas_call(
        _enc_h_body,
        grid=(NP // BN,),
        in_specs=[
            pl.BlockSpec((BN, x_p.shape[1]), lambda i: (i, 0)),
            pl.BlockSpec((x_p.shape[1], H), lambda i: (0, 0)),
            pl.BlockSpec((1, H), lambda i: (0, 0)),
        ],
        out_specs=pl.BlockSpec((BN, H), lambda i: (i, 0)),
        out_shape=jax.ShapeDtypeStruct((NP, H), jnp.float32),
    )(x_p, w, b.reshape(1, H))


def _enc_ea(ea_attr, w, b):
    E, DE = ea_attr.shape
    BE = 2000
    return pl.pallas_call(
        _enc_h_body,
        grid=(E // BE,),
        in_specs=[
            pl.BlockSpec((BE, DE), lambda i: (i, 0)),
            pl.BlockSpec((DE, H), lambda i: (0, 0)),
            pl.BlockSpec((1, H), lambda i: (0, 0)),
        ],
        out_specs=pl.BlockSpec((BE, H), lambda i: (i, 0)),
        out_shape=jax.ShapeDtypeStruct((E, H), jnp.float32),
    )(ea_attr, w, b.reshape(1, H))


def _layer_dense_body(aggr_ref, hv_ref, hp_ref, w1_ref, b1_ref, g1_ref,
                      be1_ref, w2_ref, b2_ref, ng_ref, nb_ref,
                      hn_ref, r_ref):
    o = aggr_ref[...] + hv_ref[...]
    u = jnp.dot(o, w1_ref[...], preferred_element_type=jnp.float32) + b1_ref[...]
    m = jnp.mean(u, axis=-1, keepdims=True)
    v = jnp.mean((u - m) ** 2, axis=-1, keepdims=True)
    u = (u - m) * lax.rsqrt(v + 1e-5) * g1_ref[...] + be1_ref[...]
    u = jnp.maximum(u, 0.0)
    y = jnp.dot(u, w2_ref[...], preferred_element_type=jnp.float32) + b2_ref[...]
    hn = hp_ref[...] + y
    hn_ref[...] = hn
    m2 = jnp.mean(hn, axis=-1, keepdims=True)
    v2 = jnp.mean((hn - m2) ** 2, axis=-1, keepdims=True)
    r_ref[...] = jnp.maximum(
        (hn - m2) * lax.rsqrt(v2 + 1e-5) * ng_ref[...] + nb_ref[...], 0.0)


def _layer_dense(aggr, hv, hp, cp, ng, nb):
    BN = 1280
    H2 = 2 * H
    row = lambda a, n: a.reshape(1, n)
    return pl.pallas_call(
        _layer_dense_body,
        grid=(NP // BN,),
        in_specs=[
            pl.BlockSpec((BN, H), lambda i: (i, 0)),
            pl.BlockSpec((BN, H), lambda i: (i, 0)),
            pl.BlockSpec((BN, H), lambda i: (i, 0)),
            pl.BlockSpec((H, H2), lambda i: (0, 0)),
            pl.BlockSpec((1, H2), lambda i: (0, 0)),
            pl.BlockSpec((1, H2), lambda i: (0, 0)),
            pl.BlockSpec((1, H2), lambda i: (0, 0)),
            pl.BlockSpec((H2, H), lambda i: (0, 0)),
            pl.BlockSpec((1, H), lambda i: (0, 0)),
            pl.BlockSpec((1, H), lambda i: (0, 0)),
            pl.BlockSpec((1, H), lambda i: (0, 0)),
        ],
        out_specs=[
            pl.BlockSpec((BN, H), lambda i: (i, 0)),
            pl.BlockSpec((BN, H), lambda i: (i, 0)),
        ],
        out_shape=[
            jax.ShapeDtypeStruct((NP, H), jnp.float32),
            jax.ShapeDtypeStruct((NP, H), jnp.float32),
        ],
    )(aggr, hv, hp, cp["W1"], row(cp["b1"], H2), row(cp["g1"], H2),
      row(cp["be1"], H2), cp["W2"], row(cp["b2"], H), row(ng, H), row(nb, H))


def _ab_body(hf_ref, w1a_ref, w1b_ref, b1_ref, a_ref, b_ref):
    hf = hf_ref[...]
    a_ref[...] = (jnp.dot(hf, w1a_ref[...],
                          preferred_element_type=jnp.float32) + b1_ref[...])
    b_ref[...] = jnp.dot(hf, w1b_ref[...], preferred_element_type=jnp.float32)


def _ab(hf, w1a, w1b, b1):
    BN = 1280
    return pl.pallas_call(
        _ab_body,
        grid=(NP // BN,),
        in_specs=[
            pl.BlockSpec((BN, H), lambda i: (i, 0)),
            pl.BlockSpec((H, H), lambda i: (0, 0)),
            pl.BlockSpec((H, H), lambda i: (0, 0)),
            pl.BlockSpec((1, H), lambda i: (0, 0)),
        ],
        out_specs=[
            pl.BlockSpec((BN, H), lambda i: (i, 0)),
            pl.BlockSpec((BN, H), lambda i: (i, 0)),
        ],
        out_shape=[
            jax.ShapeDtypeStruct((NP, H), jnp.float32),
            jax.ShapeDtypeStruct((NP, H), jnp.float32),
        ],
    )(hf, w1a, w1b, b1.reshape(1, H))


def kernel(x, edge_index, edge_attr, params):
    N = x.shape[0]
    E = edge_index.shape[1]
    layers = params["layers"]

    srcl, eidl, dstl, cnts = _build_lists(edge_index)

    x_p = jnp.pad(x, ((0, NP - N), (0, 0)))
    h0 = _enc_h(x_p, params["node_enc"]["W"], params["node_enc"]["b"])
    ea = _enc_ea(edge_attr, params["edge_enc"]["W"], params["edge_enc"]["b"])

    zeros_n = jnp.zeros((NP, H), jnp.float32)
    hv = h0
    hp = zeros_n
    for i in range(3):
        cp = layers[i]["conv"]
        nxt = layers[i + 1] if i < 2 else layers[0]
        aggr = _sc_layer(hv, ea, srcl, eidl, dstl, cnts, cp["t"])
        hp, hv = _layer_dense(aggr, hv, hp, cp,
                              nxt["norm_g"], nxt["norm_b"])
    hf = hv

    m = params["mlp"]
    a_t, b_t = _ab(hf, m["W1"][:H, :], m["W1"][H:, :], m["b1"])

    EP = ((E + 64 * NW - 1) // (64 * NW)) * (64 * NW)
    ei_p = jnp.pad(edge_index, ((0, 0), (0, EP - E)))
    fp = jnp.zeros((4, H), jnp.float32)
    fp = fp.at[1].set(m["W2"][:, 0])
    fp = fp.at[2].set(m["W2"][:, 1])
    fp = fp.at[3, 0:2].set(m["b2"])
    out = _sc_final(a_t, b_t, ei_p, fp)
    return out[:E]


# speedup vs baseline: 1.2113x; 1.0173x over previous
import functools

import jax
import jax.numpy as jnp
from jax import lax
from jax.experimental import pallas as pl
from jax.experimental.pallas import tpu as pltpu
from jax.experimental.pallas import tpu_sc as plsc

NC, NS, LANES = 2, 16, 16
NW = NC * NS
NP = 10240
RPT = NP // NW
TROWS = RPT + 1
H = 256
Q = 64
BE = 128
FLUSH = 8192
LBUF = FLUSH + 272
ECAP = 19 * FLUSH + LBUF + 64

_mesh = plsc.VectorSubcoreMesh(core_axis_name="c", subcore_axis_name="s")
_SC_PARAMS = pltpu.CompilerParams(needs_layout_passes=False)
_SC_PARAMS_NT = pltpu.CompilerParams(needs_layout_passes=False,
                                     use_tc_tiling_on_sc=False)


def _wid():
    return lax.axis_index("s") * NC + lax.axis_index("c")


def _build_body(ei, srcl, eidl, dstl, cnts, stage_s, stage_d,
                bufs, bufe, bufd, cw):
    E = ei.shape[0] // 2
    BLK = 2000
    nblk = E // BLK
    wid = _wid()
    n0 = wid * RPT
    n1 = n0 + RPT
    lb = wid * ECAP
    iota = lax.broadcasted_iota(jnp.int32, (LANES,), 0)

    def blk_body(blk, carry):
        cnt, wr = carry
        pltpu.sync_copy(ei.at[pl.ds(blk * BLK, BLK)], stage_s)
        pltpu.sync_copy(ei.at[pl.ds(E + blk * BLK, BLK)], stage_d)

        def chunk_body(ci, carry2):
            cnt, wr = carry2
            d = stage_d[pl.ds(ci * LANES, LANES)]
            s = stage_s[pl.ds(ci * LANES, LANES)]
            e = blk * BLK + ci * LANES + iota
            m = (d >= n0) & (d < n1)
            mi = m.astype(jnp.int32)
            pos = cnt + plsc.cumsum(mi) - mi
            plsc.store_scatter(bufd, [pos], d - n0, mask=m)
            plsc.store_scatter(bufs, [pos], s, mask=m)
            plsc.store_scatter(bufe, [pos], e, mask=m)
            cnt = cnt + jnp.sum(mi)

            def do_flush(c_w):
                c, w = c_w
                w = pl.multiple_of(w, FLUSH)
                pltpu.sync_copy(bufd.at[pl.ds(0, FLUSH)],
                                dstl.at[pl.ds(lb + w, FLUSH)])
                pltpu.sync_copy(bufs.at[pl.ds(0, FLUSH)],
                                srcl.at[pl.ds(lb + w, FLUSH)])
                pltpu.sync_copy(bufe.at[pl.ds(0, FLUSH)],
                                eidl.at[pl.ds(lb + w, FLUSH)])
                td = bufd[pl.ds(FLUSH, LANES)]
                ts = bufs[pl.ds(FLUSH, LANES)]
                te = bufe[pl.ds(FLUSH, LANES)]
                bufd[pl.ds(0, LANES)] = td
                bufs[pl.ds(0, LANES)] = ts
                bufe[pl.ds(0, LANES)] = te
                return (c - FLUSH, w + FLUSH)

            return lax.cond(cnt >= FLUSH, do_flush, lambda c_w: c_w,
                            (cnt, wr))

        return lax.fori_loop(0, BLK // LANES, chunk_body, (cnt, wr))

    cnt, wr = lax.fori_loop(0, nblk, blk_body, (0, 0))
    wr = pl.multiple_of(wr, FLUSH)

    sent_d = jnp.full((LANES,), RPT, jnp.int32)
    zer = jnp.zeros((LANES,), jnp.int32)
    for i in range(16):
        bufd[pl.ds(cnt + i * LANES, LANES)] = sent_d
        bufs[pl.ds(cnt + i * LANES, LANES)] = zer
        bufe[pl.ds(cnt + i * LANES, LANES)] = zer
    total = wr + cnt
    padded = ((total + 255) // 256) * 256
    pltpu.sync_copy(bufd.at[pl.ds(0, LBUF)], dstl.at[pl.ds(lb + wr, LBUF)])
    pltpu.sync_copy(bufs.at[pl.ds(0, LBUF)], srcl.at[pl.ds(lb + wr, LBUF)])
    pltpu.sync_copy(bufe.at[pl.ds(0, LBUF)], eidl.at[pl.ds(lb + wr, LBUF)])
    cw[pl.ds(0, LANES)] = jnp.full((LANES,), padded, jnp.int32)
    pltpu.sync_copy(cw, cnts.at[pl.ds(wid * LANES, LANES)])


def _build_lists(ei):
    E = ei.shape[1]
    out_type = [
        jax.ShapeDtypeStruct((NW * ECAP,), jnp.int32),
        jax.ShapeDtypeStruct((NW * ECAP,), jnp.int32),
        jax.ShapeDtypeStruct((NW * ECAP,), jnp.int32),
        jax.ShapeDtypeStruct((NW * LANES,), jnp.int32),
    ]
    scratch = [
        pltpu.VMEM((2000,), jnp.int32),
        pltpu.VMEM((2000,), jnp.int32),
        pltpu.VMEM((LBUF,), jnp.int32),
        pltpu.VMEM((LBUF,), jnp.int32),
        pltpu.VMEM((LBUF,), jnp.int32),
        pltpu.VMEM((LANES,), jnp.int32),
    ]
    return pl.kernel(_build_body, out_type=out_type, mesh=_mesh,
                     scratch_types=scratch,
                     compiler_params=_SC_PARAMS)(ei.reshape(2 * E))


def _layer_body(hv4, ea4, srcl, eidl, dstl, cnts, tvec, aggr,
                tab_m, tab_d, tab_n,
                lbs, lbe, lbd,
                dstb0, dstb1, idxh0, idxh1, idxe0, idxe1,
                hq0, hq1, eq0, eq1, cbuf, tbuf,
                semh0, semh1, seme0, seme1):
    wid = _wid()
    n0 = wid * RPT
    lb = wid * ECAP
    pltpu.sync_copy(cnts, cbuf)
    pltpu.sync_copy(tvec, tbuf)
    count = cbuf[pl.ds(wid * LANES, LANES)][0]
    nb = count // BE
    neg = jnp.full((LANES,), -3e38, jnp.float32)
    zf = jnp.zeros((LANES,), jnp.float32)
    slots = ((dstb0, idxh0, idxe0, hq0, eq0, semh0, seme0),
             (dstb1, idxh1, idxe1, hq1, eq1, semh1, seme1))

    def quarter_body(q, _):
        def init_body(i, _):
            tab_m[i, pl.ds(0, LANES)] = neg
            tab_m[i, pl.ds(LANES, LANES)] = neg
            tab_m[i, pl.ds(2 * LANES, LANES)] = neg
            tab_m[i, pl.ds(3 * LANES, LANES)] = neg
            for kk in range(4):
                tab_d[i, pl.ds(kk * LANES, LANES)] = zf
                tab_n[i, pl.ds(kk * LANES, LANES)] = zf
            return 0
        lax.fori_loop(0, TROWS, init_body, 0)

        def load_block(j):
            boff = lb + j * (8 * BE)
            pltpu.sync_copy(srcl.at[pl.ds(boff, 8 * BE)], lbs)
            pltpu.sync_copy(eidl.at[pl.ds(boff, 8 * BE)], lbe)
            pltpu.sync_copy(dstl.at[pl.ds(boff, 8 * BE)], lbd)

        def issue(bn, slot):
            dstb, idxh, idxe, hq, eq, semh, seme = slot
            o = (bn % 8) * BE
            for c in range(BE // LANES):
                v = lbs[pl.ds(o + c * LANES, LANES)]
                idxh[pl.ds(c * LANES, LANES)] = v * 4 + q
                w = lbe[pl.ds(o + c * LANES, LANES)]
                idxe[pl.ds(c * LANES, LANES)] = w * 4 + q
                dstb[pl.ds(c * LANES, LANES)] = lbd[pl.ds(o + c * LANES,
                                                          LANES)]
            pltpu.async_copy(hv4.at[idxh], hq, semh)
            pltpu.async_copy(ea4.at[idxe], eq, seme)

        def compute(slot):
            dstb, idxh, idxe, hq, eq, semh, seme = slot
            pltpu.make_async_copy(hv4.at[idxh], hq, semh).wait()
            pltpu.make_async_copy(ea4.at[idxe], eq, seme).wait()
            tv = tbuf[pl.ds(0, LANES)]

            def grp_body(g, _):
                dvec = dstb[pl.ds(g * LANES, LANES)]
                for jj in range(LANES):
                    j = g * LANES + jj
                    r = dvec[jj]
                    for k in range(4):
                        hh = hq[j, pl.ds(k * LANES, LANES)]
                        ee = eq[j, pl.ds(k * LANES, LANES)]
                        msg = jnp.maximum(hh + ee, 0.0) + 1e-7
                        s = msg * tv
                        mo = tab_m[r, pl.ds(k * LANES, LANES)]
                        mn = jnp.maximum(mo, s)
                        e1 = jnp.exp(s - mn)
                        sc = jnp.exp(mo - mn)
                        dd = tab_d[r, pl.ds(k * LANES, LANES)]
                        nn = tab_n[r, pl.ds(k * LANES, LANES)]
                        tab_d[r, pl.ds(k * LANES, LANES)] = dd * sc + e1
                        tab_n[r, pl.ds(k * LANES, LANES)] = nn * sc + msg * e1
                        tab_m[r, pl.ds(k * LANES, LANES)] = mn
                return 0
            lax.fori_loop(0, BE // LANES, grp_body, 0)

        @pl.when(nb > 0)
        def _():
            load_block(0)
            issue(0, slots[0])

        @pl.when(nb > 1)
        def _():
            issue(1, slots[1])

        def pair_body(i, _):
            compute(slots[0])
            bn = 2 * i + 2

            @pl.when(bn < nb)
            def _():
                @pl.when(bn % 8 == 0)
                def _():
                    load_block(bn // 8)
                issue(bn, slots[0])

            compute(slots[1])

            @pl.when(bn + 1 < nb)
            def _():
                issue(bn + 1, slots[1])
            return 0
        lax.fori_loop(0, nb // 2, pair_body, 0)

        def aggr_body(i, _):
            for kk in range(4):
                dd = tab_d[i, pl.ds(kk * LANES, LANES)]
                nn = tab_n[i, pl.ds(kk * LANES, LANES)]
                tab_n[i, pl.ds(kk * LANES, LANES)] = nn / (dd + 1e-16)
            return 0
        lax.fori_loop(0, RPT, aggr_body, 0)
        pltpu.sync_copy(tab_n.at[pl.ds(0, RPT), :],
                        aggr.at[pl.ds(n0, RPT), pl.ds(q * Q, Q)])
        return 0

    lax.fori_loop(0, 4, quarter_body, 0)


def _sc_layer(hv, ea, srcl, eidl, dstl, cnts, t):
    E = ea.shape[0]
    hv4 = hv.reshape(4 * NP, Q)
    ea4 = ea.reshape(4 * E, Q)
    tvec = jnp.full((LANES,), t, jnp.float32)
    scratch = [
        pltpu.VMEM((TROWS, Q), jnp.float32),
        pltpu.VMEM((TROWS, Q), jnp.float32),
        pltpu.VMEM((TROWS, Q), jnp.float32),
        pltpu.VMEM((8 * BE,), jnp.int32),
        pltpu.VMEM((8 * BE,), jnp.int32),
        pltpu.VMEM((8 * BE,), jnp.int32),
        pltpu.VMEM((BE,), jnp.int32),
        pltpu.VMEM((BE,), jnp.int32),
        pltpu.VMEM((BE,), jnp.int32),
        pltpu.VMEM((BE,), jnp.int32),
        pltpu.VMEM((BE,), jnp.int32),
        pltpu.VMEM((BE,), jnp.int32),
        pltpu.VMEM((BE, Q), jnp.float32),
        pltpu.VMEM((BE, Q), jnp.float32),
        pltpu.VMEM((BE, Q), jnp.float32),
        pltpu.VMEM((BE, Q), jnp.float32),
        pltpu.VMEM((NW * LANES,), jnp.int32),
        pltpu.VMEM((LANES,), jnp.float32),
        pltpu.SemaphoreType.DMA,
        pltpu.SemaphoreType.DMA,
        pltpu.SemaphoreType.DMA,
        pltpu.SemaphoreType.DMA,
    ]
    return pl.kernel(_layer_body,
                     out_type=jax.ShapeDtypeStruct((NP, H), jnp.float32),
                     mesh=_mesh, scratch_types=scratch,
                     compiler_params=_SC_PARAMS_NT)(
                         hv4, ea4, srcl, eidl, dstl, cnts, tvec)


def _final_body(a_t, b_t, ei, fp, out, srcb0, srcb1, dstb0, dstb1,
                ha0, ha1, hb0, hb1, fbuf, outb, sema0, sema1, semb0, semb1):
    Ep = ei.shape[0] // 2
    ept = Ep // NW
    wid = _wid()
    e0 = wid * ept
    pltpu.sync_copy(fp, fbuf)
    nb = ept // 64
    slots = ((srcb0, dstb0, ha0, hb0, sema0, semb0),
             (srcb1, dstb1, ha1, hb1, sema1, semb1))
    iota = lax.broadcasted_iota(jnp.int32, (LANES,), 0)

    def issue(bn, slot):
        srcb, dstb, ha, hb, sema, semb = slot
        off = bn * 64
        pltpu.sync_copy(ei.at[pl.ds(e0 + off, 64)], srcb)
        pltpu.sync_copy(ei.at[pl.ds(Ep + e0 + off, 64)], dstb)
        pltpu.async_copy(a_t.at[srcb], ha, sema)
        pltpu.async_copy(b_t.at[dstb], hb, semb)

    def compute(bn, slot):
        srcb, dstb, ha, hb, sema, semb = slot
        pltpu.make_async_copy(a_t.at[srcb], ha, sema).wait()
        pltpu.make_async_copy(b_t.at[dstb], hb, semb).wait()
        off = bn * 64

        def grp_body(g, _):
            vec = jnp.zeros((LANES,), jnp.float32)
            for jj in range(8):
                j = g * 8 + jj
                acc0 = jnp.zeros((LANES,), jnp.float32)
                acc1 = jnp.zeros((LANES,), jnp.float32)
                for k in range(H // LANES):
                    z = jnp.maximum(ha[j, pl.ds(k * LANES, LANES)]
                                    + hb[j, pl.ds(k * LANES, LANES)], 0.0)
                    acc0 = acc0 + z * fbuf[1, pl.ds(k * LANES, LANES)]
                    acc1 = acc1 + z * fbuf[2, pl.ds(k * LANES, LANES)]
                b2v = fbuf[3, pl.ds(0, LANES)]
                o0 = jnp.sum(acc0) + b2v[0]
                o1 = jnp.sum(acc1) + b2v[1]
                vec = (vec + jnp.where(iota == 2 * jj, o0, 0.0)
                       + jnp.where(iota == 2 * jj + 1, o1, 0.0))
            outb[pl.ds((off + g * 8) * 2, LANES)] = vec
            return 0
        lax.fori_loop(0, 8, grp_body, 0)

    issue(0, slots[0])
    issue(1, slots[1])

    def pair_body(i, _):
        compute(2 * i, slots[0])

        @pl.when(2 * i + 2 < nb)
        def _():
            issue(2 * i + 2, slots[0])

        compute(2 * i + 1, slots[1])

        @pl.when(2 * i + 3 < nb)
        def _():
            issue(2 * i + 3, slots[1])
        return 0
    lax.fori_loop(0, nb // 2, pair_body, 0)
    pltpu.sync_copy(outb, out.at[pl.ds(e0 * 2, ept * 2)])


def _sc_final(a_t, b_t, ei_p, fp):
    Ep = ei_p.shape[1]
    ept = Ep // NW
    scratch = [
        pltpu.VMEM((64,), jnp.int32),
        pltpu.VMEM((64,), jnp.int32),
        pltpu.VMEM((64,), jnp.int32),
        pltpu.VMEM((64,), jnp.int32),
        pltpu.VMEM((64, H), jnp.float32),
        pltpu.VMEM((64, H), jnp.float32),
        pltpu.VMEM((64, H), jnp.float32),
        pltpu.VMEM((64, H), jnp.float32),
        pltpu.VMEM((4, H), jnp.float32),
        pltpu.VMEM((ept * 2,), jnp.float32),
        pltpu.SemaphoreType.DMA,
        pltpu.SemaphoreType.DMA,
        pltpu.SemaphoreType.DMA,
        pltpu.SemaphoreType.DMA,
    ]
    out = pl.kernel(_final_body,
                    out_type=jax.ShapeDtypeStruct((Ep * 2,), jnp.float32),
                    mesh=_mesh, scratch_types=scratch,
                    compiler_params=_SC_PARAMS)(
                        a_t, b_t, ei_p.reshape(2 * Ep), fp)
    return out.reshape(Ep, 2)


def _enc_h_body(x_ref, w_ref, b_ref, o_ref):
    o_ref[...] = (jnp.dot(x_ref[...], w_ref[...],
                          preferred_element_type=jnp.float32) + b_ref[...])


def _enc_h(x_p, w, b):
    BN = 1280
    return pl.pallas_call(
        _enc_h_body,
        grid=(NP // BN,),
        in_specs=[
            pl.BlockSpec((BN, x_p.shape[1]), lambda i: (i, 0)),
            pl.BlockSpec((x_p.shape[1], H), lambda i: (0, 0)),
            pl.BlockSpec((1, H), lambda i: (0, 0)),
        ],
        out_specs=pl.BlockSpec((BN, H), lambda i: (i, 0)),
        out_shape=jax.ShapeDtypeStruct((NP, H), jnp.float32),
    )(x_p, w, b.reshape(1, H))


def _enc_ea(ea_attr, w, b):
    E, DE = ea_attr.shape
    BE = 2000
    return pl.pallas_call(
        _enc_h_body,
        grid=(E // BE,),
        in_specs=[
            pl.BlockSpec((BE, DE), lambda i: (i, 0)),
            pl.BlockSpec((DE, H), lambda i: (0, 0)),
            pl.BlockSpec((1, H), lambda i: (0, 0)),
        ],
        out_specs=pl.BlockSpec((BE, H), lambda i: (i, 0)),
        out_shape=jax.ShapeDtypeStruct((E, H), jnp.float32),
    )(ea_attr, w, b.reshape(1, H))


def _layer_dense_body(aggr_ref, hv_ref, hp_ref, w1_ref, b1_ref, g1_ref,
                      be1_ref, w2_ref, b2_ref, ng_ref, nb_ref,
                      hn_ref, r_ref):
    o = aggr_ref[...] + hv_ref[...]
    u = jnp.dot(o, w1_ref[...], preferred_element_type=jnp.float32) + b1_ref[...]
    m = jnp.mean(u, axis=-1, keepdims=True)
    v = jnp.mean((u - m) ** 2, axis=-1, keepdims=True)
    u = (u - m) * lax.rsqrt(v + 1e-5) * g1_ref[...] + be1_ref[...]
    u = jnp.maximum(u, 0.0)
    y = jnp.dot(u, w2_ref[...], preferred_element_type=jnp.float32) + b2_ref[...]
    hn = hp_ref[...] + y
    hn_ref[...] = hn
    m2 = jnp.mean(hn, axis=-1, keepdims=True)
    v2 = jnp.mean((hn - m2) ** 2, axis=-1, keepdims=True)
    r_ref[...] = jnp.maximum(
        (hn - m2) * lax.rsqrt(v2 + 1e-5) * ng_ref[...] + nb_ref[...], 0.0)


def _layer_dense(aggr, hv, hp, cp, ng, nb):
    BN = 1280
    H2 = 2 * H
    row = lambda a, n: a.reshape(1, n)
    return pl.pallas_call(
        _layer_dense_body,
        grid=(NP // BN,),
        in_specs=[
            pl.BlockSpec((BN, H), lambda i: (i, 0)),
            pl.BlockSpec((BN, H), lambda i: (i, 0)),
            pl.BlockSpec((BN, H), lambda i: (i, 0)),
            pl.BlockSpec((H, H2), lambda i: (0, 0)),
            pl.BlockSpec((1, H2), lambda i: (0, 0)),
            pl.BlockSpec((1, H2), lambda i: (0, 0)),
            pl.BlockSpec((1, H2), lambda i: (0, 0)),
            pl.BlockSpec((H2, H), lambda i: (0, 0)),
            pl.BlockSpec((1, H), lambda i: (0, 0)),
            pl.BlockSpec((1, H), lambda i: (0, 0)),
            pl.BlockSpec((1, H), lambda i: (0, 0)),
        ],
        out_specs=[
            pl.BlockSpec((BN, H), lambda i: (i, 0)),
            pl.BlockSpec((BN, H), lambda i: (i, 0)),
        ],
        out_shape=[
            jax.ShapeDtypeStruct((NP, H), jnp.float32),
            jax.ShapeDtypeStruct((NP, H), jnp.float32),
        ],
    )(aggr, hv, hp, cp["W1"], row(cp["b1"], H2), row(cp["g1"], H2),
      row(cp["be1"], H2), cp["W2"], row(cp["b2"], H), row(ng, H), row(nb, H))


def _ab_body(hf_ref, w1a_ref, w1b_ref, b1_ref, a_ref, b_ref):
    hf = hf_ref[...]
    a_ref[...] = (jnp.dot(hf, w1a_ref[...],
                          preferred_element_type=jnp.float32) + b1_ref[...])
    b_ref[...] = jnp.dot(hf, w1b_ref[...], preferred_element_type=jnp.float32)


def _ab(hf, w1a, w1b, b1):
    BN = 1280
    return pl.pallas_call(
        _ab_body,
        grid=(NP // BN,),
        in_specs=[
            pl.BlockSpec((BN, H), lambda i: (i, 0)),
            pl.BlockSpec((H, H), lambda i: (0, 0)),
            pl.BlockSpec((H, H), lambda i: (0, 0)),
            pl.BlockSpec((1, H), lambda i: (0, 0)),
        ],
        out_specs=[
            pl.BlockSpec((BN, H), lambda i: (i, 0)),
            pl.BlockSpec((BN, H), lambda i: (i, 0)),
        ],
        out_shape=[
            jax.ShapeDtypeStruct((NP, H), jnp.float32),
            jax.ShapeDtypeStruct((NP, H), jnp.float32),
        ],
    )(hf, w1a, w1b, b1.reshape(1, H))


def kernel(x, edge_index, edge_attr, params):
    N = x.shape[0]
    E = edge_index.shape[1]
    layers = params["layers"]

    srcl, eidl, dstl, cnts = _build_lists(edge_index)

    x_p = jnp.pad(x, ((0, NP - N), (0, 0)))
    h0 = _enc_h(x_p, params["node_enc"]["W"], params["node_enc"]["b"])
    ea = _enc_ea(edge_attr, params["edge_enc"]["W"], params["edge_enc"]["b"])

    zeros_n = jnp.zeros((NP, H), jnp.float32)
    hv = h0
    hp = zeros_n
    for i in range(3):
        cp = layers[i]["conv"]
        nxt = layers[i + 1] if i < 2 else layers[0]
        aggr = _sc_layer(hv, ea, srcl, eidl, dstl, cnts, cp["t"])
        hp, hv = _layer_dense(aggr, hv, hp, cp,
                              nxt["norm_g"], nxt["norm_b"])
    hf = hv

    m = params["mlp"]
    a_t, b_t = _ab(hf, m["W1"][:H, :], m["W1"][H:, :], m["b1"])

    EP = ((E + 128 * NW - 1) // (128 * NW)) * (128 * NW)
    ei_p = jnp.pad(edge_index, ((0, 0), (0, EP - E)))
    fp = jnp.zeros((4, H), jnp.float32)
    fp = fp.at[1].set(m["W2"][:, 0])
    fp = fp.at[2].set(m["W2"][:, 1])
    fp = fp.at[3, 0:2].set(m["b2"])
    out = _sc_final(a_t, b_t, ei_p, fp)
    return out[:E]


# speedup vs baseline: 3.7463x; 3.0926x over previous
import functools

import jax
import jax.numpy as jnp
from jax import lax
from jax.experimental import pallas as pl
from jax.experimental.pallas import tpu as pltpu
from jax.experimental.pallas import tpu_sc as plsc

NC, NS, LANES = 2, 16, 16
NW = NC * NS
NP = 10240
RPT = NP // NW
TROWS = RPT + 1
H = 256
Q = 64
BE = 128
FLUSH = 8192
LBUF = FLUSH + 272
ECAP = 19 * FLUSH + LBUF + 64

_mesh = plsc.VectorSubcoreMesh(core_axis_name="c", subcore_axis_name="s")
_SC_PARAMS = pltpu.CompilerParams(needs_layout_passes=False)
_SC_PARAMS_NT = pltpu.CompilerParams(needs_layout_passes=False,
                                     use_tc_tiling_on_sc=False)


def _wid():
    return lax.axis_index("s") * NC + lax.axis_index("c")


def _build_body(ei, srcl, eidl, dstl, cnts, stage_s, stage_d,
                bufs, bufe, bufd, cw):
    E = ei.shape[0] // 2
    BLK = 2000
    nblk = E // BLK
    wid = _wid()
    n0 = wid * RPT
    n1 = n0 + RPT
    lb = wid * ECAP
    iota = lax.broadcasted_iota(jnp.int32, (LANES,), 0)

    def blk_body(blk, carry):
        cnt, wr = carry
        pltpu.sync_copy(ei.at[pl.ds(blk * BLK, BLK)], stage_s)
        pltpu.sync_copy(ei.at[pl.ds(E + blk * BLK, BLK)], stage_d)

        def chunk_body(ci, carry2):
            cnt, wr = carry2
            d = stage_d[pl.ds(ci * LANES, LANES)]
            s = stage_s[pl.ds(ci * LANES, LANES)]
            e = blk * BLK + ci * LANES + iota
            m = (d >= n0) & (d < n1)
            mi = m.astype(jnp.int32)
            pos = cnt + plsc.cumsum(mi) - mi
            plsc.store_scatter(bufd, [pos], d - n0, mask=m)
            plsc.store_scatter(bufs, [pos], s, mask=m)
            plsc.store_scatter(bufe, [pos], e, mask=m)
            cnt = cnt + jnp.sum(mi)

            def do_flush(c_w):
                c, w = c_w
                w = pl.multiple_of(w, FLUSH)
                pltpu.sync_copy(bufd.at[pl.ds(0, FLUSH)],
                                dstl.at[pl.ds(lb + w, FLUSH)])
                pltpu.sync_copy(bufs.at[pl.ds(0, FLUSH)],
                                srcl.at[pl.ds(lb + w, FLUSH)])
                pltpu.sync_copy(bufe.at[pl.ds(0, FLUSH)],
                                eidl.at[pl.ds(lb + w, FLUSH)])
                td = bufd[pl.ds(FLUSH, LANES)]
                ts = bufs[pl.ds(FLUSH, LANES)]
                te = bufe[pl.ds(FLUSH, LANES)]
                bufd[pl.ds(0, LANES)] = td
                bufs[pl.ds(0, LANES)] = ts
                bufe[pl.ds(0, LANES)] = te
                return (c - FLUSH, w + FLUSH)

            return lax.cond(cnt >= FLUSH, do_flush, lambda c_w: c_w,
                            (cnt, wr))

        return lax.fori_loop(0, BLK // LANES, chunk_body, (cnt, wr))

    cnt, wr = lax.fori_loop(0, nblk, blk_body, (0, 0))
    wr = pl.multiple_of(wr, FLUSH)

    sent_d = jnp.full((LANES,), RPT, jnp.int32)
    zer = jnp.zeros((LANES,), jnp.int32)
    for i in range(16):
        bufd[pl.ds(cnt + i * LANES, LANES)] = sent_d
        bufs[pl.ds(cnt + i * LANES, LANES)] = zer
        bufe[pl.ds(cnt + i * LANES, LANES)] = zer
    total = wr + cnt
    padded = ((total + 255) // 256) * 256
    pltpu.sync_copy(bufd.at[pl.ds(0, LBUF)], dstl.at[pl.ds(lb + wr, LBUF)])
    pltpu.sync_copy(bufs.at[pl.ds(0, LBUF)], srcl.at[pl.ds(lb + wr, LBUF)])
    pltpu.sync_copy(bufe.at[pl.ds(0, LBUF)], eidl.at[pl.ds(lb + wr, LBUF)])
    cw[pl.ds(0, LANES)] = jnp.full((LANES,), padded, jnp.int32)
    pltpu.sync_copy(cw, cnts.at[pl.ds(wid * LANES, LANES)])


def _build_lists(ei):
    E = ei.shape[1]
    out_type = [
        jax.ShapeDtypeStruct((NW * ECAP,), jnp.int32),
        jax.ShapeDtypeStruct((NW * ECAP,), jnp.int32),
        jax.ShapeDtypeStruct((NW * ECAP,), jnp.int32),
        jax.ShapeDtypeStruct((NW * LANES,), jnp.int32),
    ]
    scratch = [
        pltpu.VMEM((2000,), jnp.int32),
        pltpu.VMEM((2000,), jnp.int32),
        pltpu.VMEM((LBUF,), jnp.int32),
        pltpu.VMEM((LBUF,), jnp.int32),
        pltpu.VMEM((LBUF,), jnp.int32),
        pltpu.VMEM((LANES,), jnp.int32),
    ]
    return pl.kernel(_build_body, out_type=out_type, mesh=_mesh,
                     scratch_types=scratch,
                     compiler_params=_SC_PARAMS)(ei.reshape(2 * E))


def _layer_body(hv4, ea4, srcl, eidl, dstl, cnts, tvec, aggr,
                tm0, tm1, tm2, tm3, td0, td1, td2, td3,
                tn0, tn1, tn2, tn3,
                lbs, lbe, lbd,
                dstb0, dstb1, idxh0, idxh1, idxe0, idxe1,
                hq0, hq1, eq0, eq1, outq, cbuf, tbuf,
                semh0, semh1, seme0, seme1):
    wid = _wid()
    n0 = wid * RPT
    lb = wid * ECAP
    pltpu.sync_copy(cnts, cbuf)
    pltpu.sync_copy(tvec, tbuf)
    count = cbuf[pl.ds(wid * LANES, LANES)][0]
    nb = count // BE
    neg = jnp.full((LANES,), -3e38, jnp.float32)
    zf = jnp.zeros((LANES,), jnp.float32)
    tms = (tm0, tm1, tm2, tm3)
    tds = (td0, td1, td2, td3)
    tns = (tn0, tn1, tn2, tn3)
    slots = ((dstb0, idxh0, idxe0, hq0, eq0, semh0, seme0),
             (dstb1, idxh1, idxe1, hq1, eq1, semh1, seme1))

    def quarter_body(q, _):
        def init_body(i, _):
            for kk in range(4):
                tms[kk][i, pl.ds(0, LANES)] = neg
                tds[kk][i, pl.ds(0, LANES)] = zf
                tns[kk][i, pl.ds(0, LANES)] = zf
            return 0
        lax.fori_loop(0, TROWS, init_body, 0)

        def load_block(j):
            boff = lb + j * (8 * BE)
            pltpu.sync_copy(srcl.at[pl.ds(boff, 8 * BE)], lbs)
            pltpu.sync_copy(eidl.at[pl.ds(boff, 8 * BE)], lbe)
            pltpu.sync_copy(dstl.at[pl.ds(boff, 8 * BE)], lbd)

        def issue(bn, slot):
            dstb, idxh, idxe, hq, eq, semh, seme = slot
            o = (bn % 8) * BE
            for c in range(BE // LANES):
                v = lbs[pl.ds(o + c * LANES, LANES)]
                idxh[pl.ds(c * LANES, LANES)] = v * 4 + q
                w = lbe[pl.ds(o + c * LANES, LANES)]
                idxe[pl.ds(c * LANES, LANES)] = w * 4 + q
                dstb[pl.ds(c * LANES, LANES)] = lbd[pl.ds(o + c * LANES,
                                                          LANES)]
            pltpu.async_copy(hv4.at[idxh], hq, semh)
            pltpu.async_copy(ea4.at[idxe], eq, seme)

        def compute(slot):
            dstb, idxh, idxe, hq, eq, semh, seme = slot
            pltpu.make_async_copy(hv4.at[idxh], hq, semh).wait()
            pltpu.make_async_copy(ea4.at[idxe], eq, seme).wait()
            tv = tbuf[pl.ds(0, LANES)]

            def grp_body(g, _):
                dvec = dstb[pl.ds(g * LANES, LANES)]
                for jj in range(LANES):
                    j = g * LANES + jj
                    r = dvec[jj]
                    hh = [hq[j, pl.ds(k * LANES, LANES)] for k in range(4)]
                    ee = [eq[j, pl.ds(k * LANES, LANES)] for k in range(4)]
                    mo = [tms[k][r, pl.ds(0, LANES)] for k in range(4)]
                    dd = [tds[k][r, pl.ds(0, LANES)] for k in range(4)]
                    nn = [tns[k][r, pl.ds(0, LANES)] for k in range(4)]
                    msg = [jnp.maximum(hh[k] + ee[k], 0.0) + 1e-7
                           for k in range(4)]
                    s = [msg[k] * tv for k in range(4)]
                    mn = [jnp.maximum(mo[k], s[k]) for k in range(4)]
                    e1 = [jnp.exp(s[k] - mn[k]) for k in range(4)]
                    sc = [jnp.exp(mo[k] - mn[k]) for k in range(4)]
                    for k in range(4):
                        tds[k][r, pl.ds(0, LANES)] = dd[k] * sc[k] + e1[k]
                    for k in range(4):
                        tns[k][r, pl.ds(0, LANES)] = (nn[k] * sc[k]
                                                      + msg[k] * e1[k])
                    for k in range(4):
                        tms[k][r, pl.ds(0, LANES)] = mn[k]
                return 0
            lax.fori_loop(0, BE // LANES, grp_body, 0)

        @pl.when(nb > 0)
        def _():
            load_block(0)
            issue(0, slots[0])

        @pl.when(nb > 1)
        def _():
            issue(1, slots[1])

        def pair_body(i, _):
            compute(slots[0])
            bn = 2 * i + 2

            @pl.when(bn < nb)
            def _():
                @pl.when(bn % 8 == 0)
                def _():
                    load_block(bn // 8)
                issue(bn, slots[0])

            compute(slots[1])

            @pl.when(bn + 1 < nb)
            def _():
                issue(bn + 1, slots[1])
            return 0
        lax.fori_loop(0, nb // 2, pair_body, 0)

        def aggr_body(i, _):
            for kk in range(4):
                dd = tds[kk][i, pl.ds(0, LANES)]
                nn = tns[kk][i, pl.ds(0, LANES)]
                outq[i, pl.ds(kk * LANES, LANES)] = nn / (dd + 1e-16)
            return 0
        lax.fori_loop(0, RPT, aggr_body, 0)
        pltpu.sync_copy(outq, aggr.at[pl.ds(n0, RPT), pl.ds(q * Q, Q)])
        return 0

    lax.fori_loop(0, 4, quarter_body, 0)


def _sc_layer(hv, ea, srcl, eidl, dstl, cnts, t):
    E = ea.shape[0]
    hv4 = hv.reshape(4 * NP, Q)
    ea4 = ea.reshape(4 * E, Q)
    tvec = jnp.full((LANES,), t, jnp.float32)
    scratch = (
        [pltpu.VMEM((TROWS, LANES), jnp.float32) for _ in range(12)]
        + [
            pltpu.VMEM((8 * BE,), jnp.int32),
            pltpu.VMEM((8 * BE,), jnp.int32),
            pltpu.VMEM((8 * BE,), jnp.int32),
            pltpu.VMEM((BE,), jnp.int32),
            pltpu.VMEM((BE,), jnp.int32),
            pltpu.VMEM((BE,), jnp.int32),
            pltpu.VMEM((BE,), jnp.int32),
            pltpu.VMEM((BE,), jnp.int32),
            pltpu.VMEM((BE,), jnp.int32),
            pltpu.VMEM((BE, Q), jnp.float32),
            pltpu.VMEM((BE, Q), jnp.float32),
            pltpu.VMEM((BE, Q), jnp.float32),
            pltpu.VMEM((BE, Q), jnp.float32),
            pltpu.VMEM((RPT, Q), jnp.float32),
            pltpu.VMEM((NW * LANES,), jnp.int32),
            pltpu.VMEM((LANES,), jnp.float32),
            pltpu.SemaphoreType.DMA,
            pltpu.SemaphoreType.DMA,
            pltpu.SemaphoreType.DMA,
            pltpu.SemaphoreType.DMA,
        ]
    )
    return pl.kernel(_layer_body,
                     out_type=jax.ShapeDtypeStruct((NP, H), jnp.float32),
                     mesh=_mesh, scratch_types=scratch,
                     compiler_params=_SC_PARAMS_NT)(
                         hv4, ea4, srcl, eidl, dstl, cnts, tvec)


def _final_body(a_t, b_t, ei, fp, out, srcb0, srcb1, dstb0, dstb1,
                ha0, ha1, hb0, hb1, fbuf, outb, sema0, sema1, semb0, semb1):
    Ep = ei.shape[0] // 2
    ept = Ep // NW
    wid = _wid()
    e0 = wid * ept
    pltpu.sync_copy(fp, fbuf)
    nb = ept // 64
    slots = ((srcb0, dstb0, ha0, hb0, sema0, semb0),
             (srcb1, dstb1, ha1, hb1, sema1, semb1))
    iota = lax.broadcasted_iota(jnp.int32, (LANES,), 0)

    def issue(bn, slot):
        srcb, dstb, ha, hb, sema, semb = slot
        off = bn * 64
        pltpu.sync_copy(ei.at[pl.ds(e0 + off, 64)], srcb)
        pltpu.sync_copy(ei.at[pl.ds(Ep + e0 + off, 64)], dstb)
        pltpu.async_copy(a_t.at[srcb], ha, sema)
        pltpu.async_copy(b_t.at[dstb], hb, semb)

    def compute(bn, slot):
        srcb, dstb, ha, hb, sema, semb = slot
        pltpu.make_async_copy(a_t.at[srcb], ha, sema).wait()
        pltpu.make_async_copy(b_t.at[dstb], hb, semb).wait()
        off = bn * 64

        def grp_body(g, _):
            vec = jnp.zeros((LANES,), jnp.float32)
            for jj in range(8):
                j = g * 8 + jj
                acc0 = jnp.zeros((LANES,), jnp.float32)
                acc1 = jnp.zeros((LANES,), jnp.float32)
                for k in range(H // LANES):
                    z = jnp.maximum(ha[j, pl.ds(k * LANES, LANES)]
                                    + hb[j, pl.ds(k * LANES, LANES)], 0.0)
                    acc0 = acc0 + z * fbuf[1, pl.ds(k * LANES, LANES)]
                    acc1 = acc1 + z * fbuf[2, pl.ds(k * LANES, LANES)]
                b2v = fbuf[3, pl.ds(0, LANES)]
                o0 = jnp.sum(acc0) + b2v[0]
                o1 = jnp.sum(acc1) + b2v[1]
                vec = (vec + jnp.where(iota == 2 * jj, o0, 0.0)
                       + jnp.where(iota == 2 * jj + 1, o1, 0.0))
            outb[pl.ds((off + g * 8) * 2, LANES)] = vec
            return 0
        lax.fori_loop(0, 8, grp_body, 0)

    issue(0, slots[0])
    issue(1, slots[1])

    def pair_body(i, _):
        compute(2 * i, slots[0])

        @pl.when(2 * i + 2 < nb)
        def _():
            issue(2 * i + 2, slots[0])

        compute(2 * i + 1, slots[1])

        @pl.when(2 * i + 3 < nb)
        def _():
            issue(2 * i + 3, slots[1])
        return 0
    lax.fori_loop(0, nb // 2, pair_body, 0)
    pltpu.sync_copy(outb, out.at[pl.ds(e0 * 2, ept * 2)])


def _sc_final(a_t, b_t, ei_p, fp):
    Ep = ei_p.shape[1]
    ept = Ep // NW
    scratch = [
        pltpu.VMEM((64,), jnp.int32),
        pltpu.VMEM((64,), jnp.int32),
        pltpu.VMEM((64,), jnp.int32),
        pltpu.VMEM((64,), jnp.int32),
        pltpu.VMEM((64, H), jnp.float32),
        pltpu.VMEM((64, H), jnp.float32),
        pltpu.VMEM((64, H), jnp.float32),
        pltpu.VMEM((64, H), jnp.float32),
        pltpu.VMEM((4, H), jnp.float32),
        pltpu.VMEM((ept * 2,), jnp.float32),
        pltpu.SemaphoreType.DMA,
        pltpu.SemaphoreType.DMA,
        pltpu.SemaphoreType.DMA,
        pltpu.SemaphoreType.DMA,
    ]
    out = pl.kernel(_final_body,
                    out_type=jax.ShapeDtypeStruct((Ep * 2,), jnp.float32),
                    mesh=_mesh, scratch_types=scratch,
                    compiler_params=_SC_PARAMS)(
                        a_t, b_t, ei_p.reshape(2 * Ep), fp)
    return out.reshape(Ep, 2)


def _enc_h_body(x_ref, w_ref, b_ref, o_ref):
    o_ref[...] = (jnp.dot(x_ref[...], w_ref[...],
                          preferred_element_type=jnp.float32) + b_ref[...])


def _enc_h(x_p, w, b):
    BN = 1280
    return pl.pallas_call(
        _enc_h_body,
        grid=(NP // BN,),
        in_specs=[
            pl.BlockSpec((BN, x_p.shape[1]), lambda i: (i, 0)),
            pl.BlockSpec((x_p.shape[1], H), lambda i: (0, 0)),
            pl.BlockSpec((1, H), lambda i: (0, 0)),
        ],
        out_specs=pl.BlockSpec((BN, H), lambda i: (i, 0)),
        out_shape=jax.ShapeDtypeStruct((NP, H), jnp.float32),
    )(x_p, w, b.reshape(1, H))


def _enc_ea(ea_attr, w, b):
    E, DE = ea_attr.shape
    BE = 2000
    return pl.pallas_call(
        _enc_h_body,
        grid=(E // BE,),
        in_specs=[
            pl.BlockSpec((BE, DE), lambda i: (i, 0)),
            pl.BlockSpec((DE, H), lambda i: (0, 0)),
            pl.BlockSpec((1, H), lambda i: (0, 0)),
        ],
        out_specs=pl.BlockSpec((BE, H), lambda i: (i, 0)),
        out_shape=jax.ShapeDtypeStruct((E, H), jnp.float32),
    )(ea_attr, w, b.reshape(1, H))


def _layer_dense_body(aggr_ref, hv_ref, hp_ref, w1_ref, b1_ref, g1_ref,
                      be1_ref, w2_ref, b2_ref, ng_ref, nb_ref,
                      hn_ref, r_ref):
    o = aggr_ref[...] + hv_ref[...]
    u = jnp.dot(o, w1_ref[...], preferred_element_type=jnp.float32) + b1_ref[...]
    m = jnp.mean(u, axis=-1, keepdims=True)
    v = jnp.mean((u - m) ** 2, axis=-1, keepdims=True)
    u = (u - m) * lax.rsqrt(v + 1e-5) * g1_ref[...] + be1_ref[...]
    u = jnp.maximum(u, 0.0)
    y = jnp.dot(u, w2_ref[...], preferred_element_type=jnp.float32) + b2_ref[...]
    hn = hp_ref[...] + y
    hn_ref[...] = hn
    m2 = jnp.mean(hn, axis=-1, keepdims=True)
    v2 = jnp.mean((hn - m2) ** 2, axis=-1, keepdims=True)
    r_ref[...] = jnp.maximum(
        (hn - m2) * lax.rsqrt(v2 + 1e-5) * ng_ref[...] + nb_ref[...], 0.0)


def _layer_dense(aggr, hv, hp, cp, ng, nb):
    BN = 1280
    H2 = 2 * H
    row = lambda a, n: a.reshape(1, n)
    return pl.pallas_call(
        _layer_dense_body,
        grid=(NP // BN,),
        in_specs=[
            pl.BlockSpec((BN, H), lambda i: (i, 0)),
            pl.BlockSpec((BN, H), lambda i: (i, 0)),
            pl.BlockSpec((BN, H), lambda i: (i, 0)),
            pl.BlockSpec((H, H2), lambda i: (0, 0)),
            pl.BlockSpec((1, H2), lambda i: (0, 0)),
            pl.BlockSpec((1, H2), lambda i: (0, 0)),
            pl.BlockSpec((1, H2), lambda i: (0, 0)),
            pl.BlockSpec((H2, H), lambda i: (0, 0)),
            pl.BlockSpec((1, H), lambda i: (0, 0)),
            pl.BlockSpec((1, H), lambda i: (0, 0)),
            pl.BlockSpec((1, H), lambda i: (0, 0)),
        ],
        out_specs=[
            pl.BlockSpec((BN, H), lambda i: (i, 0)),
            pl.BlockSpec((BN, H), lambda i: (i, 0)),
        ],
        out_shape=[
            jax.ShapeDtypeStruct((NP, H), jnp.float32),
            jax.ShapeDtypeStruct((NP, H), jnp.float32),
        ],
    )(aggr, hv, hp, cp["W1"], row(cp["b1"], H2), row(cp["g1"], H2),
      row(cp["be1"], H2), cp["W2"], row(cp["b2"], H), row(ng, H), row(nb, H))


def _ab_body(hf_ref, w1a_ref, w1b_ref, b1_ref, a_ref, b_ref):
    hf = hf_ref[...]
    a_ref[...] = (jnp.dot(hf, w1a_ref[...],
                          preferred_element_type=jnp.float32) + b1_ref[...])
    b_ref[...] = jnp.dot(hf, w1b_ref[...], preferred_element_type=jnp.float32)


def _ab(hf, w1a, w1b, b1):
    BN = 1280
    return pl.pallas_call(
        _ab_body,
        grid=(NP // BN,),
        in_specs=[
            pl.BlockSpec((BN, H), lambda i: (i, 0)),
            pl.BlockSpec((H, H), lambda i: (0, 0)),
            pl.BlockSpec((H, H), lambda i: (0, 0)),
            pl.BlockSpec((1, H), lambda i: (0, 0)),
        ],
        out_specs=[
            pl.BlockSpec((BN, H), lambda i: (i, 0)),
            pl.BlockSpec((BN, H), lambda i: (i, 0)),
        ],
        out_shape=[
            jax.ShapeDtypeStruct((NP, H), jnp.float32),
            jax.ShapeDtypeStruct((NP, H), jnp.float32),
        ],
    )(hf, w1a, w1b, b1.reshape(1, H))


def kernel(x, edge_index, edge_attr, params):
    N = x.shape[0]
    E = edge_index.shape[1]
    layers = params["layers"]

    srcl, eidl, dstl, cnts = _build_lists(edge_index)

    x_p = jnp.pad(x, ((0, NP - N), (0, 0)))
    h0 = _enc_h(x_p, params["node_enc"]["W"], params["node_enc"]["b"])
    ea = _enc_ea(edge_attr, params["edge_enc"]["W"], params["edge_enc"]["b"])

    zeros_n = jnp.zeros((NP, H), jnp.float32)
    hv = h0
    hp = zeros_n
    for i in range(3):
        cp = layers[i]["conv"]
        nxt = layers[i + 1] if i < 2 else layers[0]
        aggr = _sc_layer(hv, ea, srcl, eidl, dstl, cnts, cp["t"])
        hp, hv = _layer_dense(aggr, hv, hp, cp,
                              nxt["norm_g"], nxt["norm_b"])
    hf = hv

    m = params["mlp"]
    a_t, b_t = _ab(hf, m["W1"][:H, :], m["W1"][H:, :], m["b1"])

    EP = ((E + 128 * NW - 1) // (128 * NW)) * (128 * NW)
    ei_p = jnp.pad(edge_index, ((0, 0), (0, EP - E)))
    fp = jnp.zeros((4, H), jnp.float32)
    fp = fp.at[1].set(m["W2"][:, 0])
    fp = fp.at[2].set(m["W2"][:, 1])
    fp = fp.at[3, 0:2].set(m["b2"])
    out = _sc_final(a_t, b_t, ei_p, fp)
    return out[:E]


# speedup vs baseline: 3.8478x; 1.0271x over previous
import functools

import jax
import jax.numpy as jnp
from jax import lax
from jax.experimental import pallas as pl
from jax.experimental.pallas import tpu as pltpu
from jax.experimental.pallas import tpu_sc as plsc

NC, NS, LANES = 2, 16, 16
NW = NC * NS
NP = 10240
RPT = NP // NW
TROWS = RPT + 1
H = 256
Q = 64
BE = 128
FLUSH = 8192
LBUF = FLUSH + 272
ECAP = 19 * FLUSH + LBUF + 64

_mesh = plsc.VectorSubcoreMesh(core_axis_name="c", subcore_axis_name="s")
_SC_PARAMS = pltpu.CompilerParams(needs_layout_passes=False)
_SC_PARAMS_NT = pltpu.CompilerParams(needs_layout_passes=False,
                                     use_tc_tiling_on_sc=False)


def _wid():
    return lax.axis_index("s") * NC + lax.axis_index("c")


def _build_body(ei, srcl, eidl, dstl, cnts, stage_s0, stage_d0,
                stage_s1, stage_d1, bufs, bufe, bufd, cw,
                sems0, semd0, sems1, semd1):
    E = ei.shape[0] // 2
    BLK = 2000
    nblk = E // BLK
    wid = _wid()
    n0 = wid * RPT
    n1 = n0 + RPT
    lb = wid * ECAP
    iota = lax.broadcasted_iota(jnp.int32, (LANES,), 0)
    slots = ((stage_s0, stage_d0, sems0, semd0),
             (stage_s1, stage_d1, sems1, semd1))

    def issue(blk, slot):
        ss, sd, qs, qd = slot
        pltpu.async_copy(ei.at[pl.ds(blk * BLK, BLK)], ss, qs)
        pltpu.async_copy(ei.at[pl.ds(E + blk * BLK, BLK)], sd, qd)

    def compute(blk, slot, carry):
        ss, sd, qs, qd = slot
        pltpu.make_async_copy(ei.at[pl.ds(blk * BLK, BLK)], ss, qs).wait()
        pltpu.make_async_copy(ei.at[pl.ds(E + blk * BLK, BLK)], sd,
                              qd).wait()

        def chunk_body(ci, carry2):
            cnt, wr = carry2
            d = sd[pl.ds(ci * LANES, LANES)]
            s = ss[pl.ds(ci * LANES, LANES)]
            e = blk * BLK + ci * LANES + iota
            m = (d >= n0) & (d < n1)
            mi = m.astype(jnp.int32)
            pos = cnt + plsc.cumsum(mi) - mi
            plsc.store_scatter(bufd, [pos], d - n0, mask=m)
            plsc.store_scatter(bufs, [pos], s, mask=m)
            plsc.store_scatter(bufe, [pos], e, mask=m)
            cnt = cnt + jnp.sum(mi)

            def do_flush(c_w):
                c, w = c_w
                w = pl.multiple_of(w, FLUSH)
                pltpu.sync_copy(bufd.at[pl.ds(0, FLUSH)],
                                dstl.at[pl.ds(lb + w, FLUSH)])
                pltpu.sync_copy(bufs.at[pl.ds(0, FLUSH)],
                                srcl.at[pl.ds(lb + w, FLUSH)])
                pltpu.sync_copy(bufe.at[pl.ds(0, FLUSH)],
                                eidl.at[pl.ds(lb + w, FLUSH)])
                td = bufd[pl.ds(FLUSH, LANES)]
                ts = bufs[pl.ds(FLUSH, LANES)]
                te = bufe[pl.ds(FLUSH, LANES)]
                bufd[pl.ds(0, LANES)] = td
                bufs[pl.ds(0, LANES)] = ts
                bufe[pl.ds(0, LANES)] = te
                return (c - FLUSH, w + FLUSH)

            return lax.cond(cnt >= FLUSH, do_flush, lambda c_w: c_w,
                            (cnt, wr))

        return lax.fori_loop(0, BLK // LANES, chunk_body, carry)

    issue(0, slots[0])
    issue(1, slots[1])

    def pair_body(i, carry):
        carry = compute(2 * i, slots[0], carry)

        @pl.when(2 * i + 2 < nblk)
        def _():
            issue(2 * i + 2, slots[0])

        carry = compute(2 * i + 1, slots[1], carry)

        @pl.when(2 * i + 3 < nblk)
        def _():
            issue(2 * i + 3, slots[1])
        return carry

    cnt, wr = lax.fori_loop(0, nblk // 2, pair_body, (0, 0))
    wr = pl.multiple_of(wr, FLUSH)

    sent_d = jnp.full((LANES,), RPT, jnp.int32)
    zer = jnp.zeros((LANES,), jnp.int32)
    for i in range(16):
        bufd[pl.ds(cnt + i * LANES, LANES)] = sent_d
        bufs[pl.ds(cnt + i * LANES, LANES)] = zer
        bufe[pl.ds(cnt + i * LANES, LANES)] = zer
    total = wr + cnt
    padded = ((total + 255) // 256) * 256
    pltpu.sync_copy(bufd.at[pl.ds(0, LBUF)], dstl.at[pl.ds(lb + wr, LBUF)])
    pltpu.sync_copy(bufs.at[pl.ds(0, LBUF)], srcl.at[pl.ds(lb + wr, LBUF)])
    pltpu.sync_copy(bufe.at[pl.ds(0, LBUF)], eidl.at[pl.ds(lb + wr, LBUF)])
    cw[pl.ds(0, LANES)] = jnp.full((LANES,), padded, jnp.int32)
    pltpu.sync_copy(cw, cnts.at[pl.ds(wid * LANES, LANES)])


def _build_lists(ei):
    E = ei.shape[1]
    out_type = [
        jax.ShapeDtypeStruct((NW * ECAP,), jnp.int32),
        jax.ShapeDtypeStruct((NW * ECAP,), jnp.int32),
        jax.ShapeDtypeStruct((NW * ECAP,), jnp.int32),
        jax.ShapeDtypeStruct((NW * LANES,), jnp.int32),
    ]
    scratch = [
        pltpu.VMEM((2000,), jnp.int32),
        pltpu.VMEM((2000,), jnp.int32),
        pltpu.VMEM((2000,), jnp.int32),
        pltpu.VMEM((2000,), jnp.int32),
        pltpu.VMEM((LBUF,), jnp.int32),
        pltpu.VMEM((LBUF,), jnp.int32),
        pltpu.VMEM((LBUF,), jnp.int32),
        pltpu.VMEM((LANES,), jnp.int32),
        pltpu.SemaphoreType.DMA,
        pltpu.SemaphoreType.DMA,
        pltpu.SemaphoreType.DMA,
        pltpu.SemaphoreType.DMA,
    ]
    return pl.kernel(_build_body, out_type=out_type, mesh=_mesh,
                     scratch_types=scratch,
                     compiler_params=_SC_PARAMS)(ei.reshape(2 * E))


def _layer_body(hv4, ea4, srcl, eidl, dstl, cnts, tvec, aggr,
                tm0, tm1, tm2, tm3, td0, td1, td2, td3,
                tn0, tn1, tn2, tn3,
                lbs, lbe, lbd,
                dstb0, dstb1, idxh0, idxh1, idxe0, idxe1,
                hq0, hq1, eq0, eq1, outq, cbuf, tbuf,
                semh0, semh1, seme0, seme1):
    wid = _wid()
    n0 = wid * RPT
    lb = wid * ECAP
    pltpu.sync_copy(cnts, cbuf)
    pltpu.sync_copy(tvec, tbuf)
    count = cbuf[pl.ds(wid * LANES, LANES)][0]
    nb = count // BE
    neg = jnp.full((LANES,), -3e38, jnp.float32)
    zf = jnp.zeros((LANES,), jnp.float32)
    tms = (tm0, tm1, tm2, tm3)
    tds = (td0, td1, td2, td3)
    tns = (tn0, tn1, tn2, tn3)
    slots = ((dstb0, idxh0, idxe0, hq0, eq0, semh0, seme0),
             (dstb1, idxh1, idxe1, hq1, eq1, semh1, seme1))

    def quarter_body(q, _):
        def init_body(i, _):
            for kk in range(4):
                tms[kk][i, pl.ds(0, LANES)] = neg
                tds[kk][i, pl.ds(0, LANES)] = zf
                tns[kk][i, pl.ds(0, LANES)] = zf
            return 0
        lax.fori_loop(0, TROWS, init_body, 0)

        def load_block(j):
            boff = lb + j * (8 * BE)
            pltpu.sync_copy(srcl.at[pl.ds(boff, 8 * BE)], lbs)
            pltpu.sync_copy(eidl.at[pl.ds(boff, 8 * BE)], lbe)
            pltpu.sync_copy(dstl.at[pl.ds(boff, 8 * BE)], lbd)

        def issue(bn, slot):
            dstb, idxh, idxe, hq, eq, semh, seme = slot
            o = (bn % 8) * BE
            for c in range(BE // LANES):
                v = lbs[pl.ds(o + c * LANES, LANES)]
                idxh[pl.ds(c * LANES, LANES)] = v * 4 + q
                w = lbe[pl.ds(o + c * LANES, LANES)]
                idxe[pl.ds(c * LANES, LANES)] = w * 4 + q
                dstb[pl.ds(c * LANES, LANES)] = lbd[pl.ds(o + c * LANES,
                                                          LANES)]
            pltpu.async_copy(hv4.at[idxh], hq, semh)
            pltpu.async_copy(ea4.at[idxe], eq, seme)

        def compute(slot):
            dstb, idxh, idxe, hq, eq, semh, seme = slot
            pltpu.make_async_copy(hv4.at[idxh], hq, semh).wait()
            pltpu.make_async_copy(ea4.at[idxe], eq, seme).wait()
            tv = tbuf[pl.ds(0, LANES)]

            def grp_body(g, _):
                dvec = dstb[pl.ds(g * LANES, LANES)]
                for jj in range(LANES):
                    j = g * LANES + jj
                    r = dvec[jj]
                    hh = [hq[j, pl.ds(k * LANES, LANES)] for k in range(4)]
                    ee = [eq[j, pl.ds(k * LANES, LANES)] for k in range(4)]
                    mo = [tms[k][r, pl.ds(0, LANES)] for k in range(4)]
                    dd = [tds[k][r, pl.ds(0, LANES)] for k in range(4)]
                    nn = [tns[k][r, pl.ds(0, LANES)] for k in range(4)]
                    msg = [jnp.maximum(hh[k] + ee[k], 0.0) + 1e-7
                           for k in range(4)]
                    s = [msg[k] * tv for k in range(4)]
                    mn = [jnp.maximum(mo[k], s[k]) for k in range(4)]
                    e1 = [jnp.exp(s[k] - mn[k]) for k in range(4)]
                    sc = [jnp.exp(mo[k] - mn[k]) for k in range(4)]
                    for k in range(4):
                        tds[k][r, pl.ds(0, LANES)] = dd[k] * sc[k] + e1[k]
                    for k in range(4):
                        tns[k][r, pl.ds(0, LANES)] = (nn[k] * sc[k]
                                                      + msg[k] * e1[k])
                    for k in range(4):
                        tms[k][r, pl.ds(0, LANES)] = mn[k]
                return 0
            lax.fori_loop(0, BE // LANES, grp_body, 0)

        @pl.when(nb > 0)
        def _():
            load_block(0)
            issue(0, slots[0])

        @pl.when(nb > 1)
        def _():
            issue(1, slots[1])

        def pair_body(i, _):
            compute(slots[0])
            bn = 2 * i + 2

            @pl.when(bn < nb)
            def _():
                @pl.when(bn % 8 == 0)
                def _():
                    load_block(bn // 8)
                issue(bn, slots[0])

            compute(slots[1])

            @pl.when(bn + 1 < nb)
            def _():
                issue(bn + 1, slots[1])
            return 0
        lax.fori_loop(0, nb // 2, pair_body, 0)

        def aggr_body(i, _):
            for kk in range(4):
                dd = tds[kk][i, pl.ds(0, LANES)]
                nn = tns[kk][i, pl.ds(0, LANES)]
                outq[i, pl.ds(kk * LANES, LANES)] = nn / (dd + 1e-16)
            return 0
        lax.fori_loop(0, RPT, aggr_body, 0)
        pltpu.sync_copy(outq, aggr.at[pl.ds(n0, RPT), pl.ds(q * Q, Q)])
        return 0

    lax.fori_loop(0, 4, quarter_body, 0)


def _sc_layer(hv, ea4, srcl, eidl, dstl, cnts, t):
    hv4 = hv.reshape(4 * NP, Q)
    tvec = jnp.full((LANES,), t, jnp.float32)
    scratch = (
        [pltpu.VMEM((TROWS, LANES), jnp.float32) for _ in range(12)]
        + [
            pltpu.VMEM((8 * BE,), jnp.int32),
            pltpu.VMEM((8 * BE,), jnp.int32),
            pltpu.VMEM((8 * BE,), jnp.int32),
            pltpu.VMEM((BE,), jnp.int32),
            pltpu.VMEM((BE,), jnp.int32),
            pltpu.VMEM((BE,), jnp.int32),
            pltpu.VMEM((BE,), jnp.int32),
            pltpu.VMEM((BE,), jnp.int32),
            pltpu.VMEM((BE,), jnp.int32),
            pltpu.VMEM((BE, Q), jnp.float32),
            pltpu.VMEM((BE, Q), jnp.float32),
            pltpu.VMEM((BE, Q), jnp.float32),
            pltpu.VMEM((BE, Q), jnp.float32),
            pltpu.VMEM((RPT, Q), jnp.float32),
            pltpu.VMEM((NW * LANES,), jnp.int32),
            pltpu.VMEM((LANES,), jnp.float32),
            pltpu.SemaphoreType.DMA,
            pltpu.SemaphoreType.DMA,
            pltpu.SemaphoreType.DMA,
            pltpu.SemaphoreType.DMA,
        ]
    )
    return pl.kernel(_layer_body,
                     out_type=jax.ShapeDtypeStruct((NP, H), jnp.float32),
                     mesh=_mesh, scratch_types=scratch,
                     compiler_params=_SC_PARAMS_NT)(
                         hv4, ea4, srcl, eidl, dstl, cnts, tvec)


def _final_body(a_t, b_t, ei, fp, out, srcb0, srcb1, dstb0, dstb1,
                ha0, ha1, hb0, hb1, fbuf, outb, sema0, sema1, semb0, semb1):
    Ep = ei.shape[0] // 2
    ept = Ep // NW
    wid = _wid()
    e0 = wid * ept
    pltpu.sync_copy(fp, fbuf)
    nb = ept // 64
    slots = ((srcb0, dstb0, ha0, hb0, sema0, semb0),
             (srcb1, dstb1, ha1, hb1, sema1, semb1))
    iota = lax.broadcasted_iota(jnp.int32, (LANES,), 0)

    def issue(bn, slot):
        srcb, dstb, ha, hb, sema, semb = slot
        off = bn * 64
        pltpu.sync_copy(ei.at[pl.ds(e0 + off, 64)], srcb)
        pltpu.sync_copy(ei.at[pl.ds(Ep + e0 + off, 64)], dstb)
        pltpu.async_copy(a_t.at[srcb], ha, sema)
        pltpu.async_copy(b_t.at[dstb], hb, semb)

    def compute(bn, slot):
        srcb, dstb, ha, hb, sema, semb = slot
        pltpu.make_async_copy(a_t.at[srcb], ha, sema).wait()
        pltpu.make_async_copy(b_t.at[dstb], hb, semb).wait()
        off = bn * 64

        def grp_body(g, _):
            vec = jnp.zeros((LANES,), jnp.float32)
            for jj in range(8):
                j = g * 8 + jj
                acc0 = jnp.zeros((LANES,), jnp.float32)
                acc1 = jnp.zeros((LANES,), jnp.float32)
                for k in range(H // LANES):
                    z = jnp.maximum(ha[j, pl.ds(k * LANES, LANES)]
                                    + hb[j, pl.ds(k * LANES, LANES)], 0.0)
                    acc0 = acc0 + z * fbuf[1, pl.ds(k * LANES, LANES)]
                    acc1 = acc1 + z * fbuf[2, pl.ds(k * LANES, LANES)]
                b2v = fbuf[3, pl.ds(0, LANES)]
                o0 = jnp.sum(acc0) + b2v[0]
                o1 = jnp.sum(acc1) + b2v[1]
                vec = (vec + jnp.where(iota == 2 * jj, o0, 0.0)
                       + jnp.where(iota == 2 * jj + 1, o1, 0.0))
            outb[pl.ds((off + g * 8) * 2, LANES)] = vec
            return 0
        lax.fori_loop(0, 8, grp_body, 0)

    issue(0, slots[0])
    issue(1, slots[1])

    def pair_body(i, _):
        compute(2 * i, slots[0])

        @pl.when(2 * i + 2 < nb)
        def _():
            issue(2 * i + 2, slots[0])

        compute(2 * i + 1, slots[1])

        @pl.when(2 * i + 3 < nb)
        def _():
            issue(2 * i + 3, slots[1])
        return 0
    lax.fori_loop(0, nb // 2, pair_body, 0)
    pltpu.sync_copy(outb, out.at[pl.ds(e0 * 2, ept * 2)])


def _sc_final(a_t, b_t, ei_p, fp):
    Ep = ei_p.shape[1]
    ept = Ep // NW
    scratch = [
        pltpu.VMEM((64,), jnp.int32),
        pltpu.VMEM((64,), jnp.int32),
        pltpu.VMEM((64,), jnp.int32),
        pltpu.VMEM((64,), jnp.int32),
        pltpu.VMEM((64, H), jnp.float32),
        pltpu.VMEM((64, H), jnp.float32),
        pltpu.VMEM((64, H), jnp.float32),
        pltpu.VMEM((64, H), jnp.float32),
        pltpu.VMEM((4, H), jnp.float32),
        pltpu.VMEM((ept * 2,), jnp.float32),
        pltpu.SemaphoreType.DMA,
        pltpu.SemaphoreType.DMA,
        pltpu.SemaphoreType.DMA,
        pltpu.SemaphoreType.DMA,
    ]
    out = pl.kernel(_final_body,
                    out_type=jax.ShapeDtypeStruct((Ep * 2,), jnp.float32),
                    mesh=_mesh, scratch_types=scratch,
                    compiler_params=_SC_PARAMS)(
                        a_t, b_t, ei_p.reshape(2 * Ep), fp)
    return out.reshape(Ep, 2)


def _enc_h_body(x_ref, w_ref, b_ref, o_ref):
    o_ref[...] = (jnp.dot(x_ref[...], w_ref[...],
                          preferred_element_type=jnp.float32) + b_ref[...])


def _enc_h(x_p, w, b):
    BN = 1280
    return pl.pallas_call(
        _enc_h_body,
        grid=(NP // BN,),
        in_specs=[
            pl.BlockSpec((BN, x_p.shape[1]), lambda i: (i, 0)),
            pl.BlockSpec((x_p.shape[1], H), lambda i: (0, 0)),
            pl.BlockSpec((1, H), lambda i: (0, 0)),
        ],
        out_specs=pl.BlockSpec((BN, H), lambda i: (i, 0)),
        out_shape=jax.ShapeDtypeStruct((NP, H), jnp.float32),
    )(x_p, w, b.reshape(1, H))


def _enc_ea(ea_attr, w, b):
    E, DE = ea_attr.shape
    BE = 2000
    return pl.pallas_call(
        _enc_h_body,
        grid=(E // BE,),
        in_specs=[
            pl.BlockSpec((BE, DE), lambda i: (i, 0)),
            pl.BlockSpec((DE, H), lambda i: (0, 0)),
            pl.BlockSpec((1, H), lambda i: (0, 0)),
        ],
        out_specs=pl.BlockSpec((BE, H), lambda i: (i, 0)),
        out_shape=jax.ShapeDtypeStruct((E, H), jnp.float32),
    )(ea_attr, w, b.reshape(1, H))


def _layer_dense_body(aggr_ref, hv_ref, hp_ref, w1_ref, b1_ref, g1_ref,
                      be1_ref, w2_ref, b2_ref, ng_ref, nb_ref,
                      hn_ref, r_ref):
    o = aggr_ref[...] + hv_ref[...]
    u = jnp.dot(o, w1_ref[...], preferred_element_type=jnp.float32) + b1_ref[...]
    m = jnp.mean(u, axis=-1, keepdims=True)
    v = jnp.mean((u - m) ** 2, axis=-1, keepdims=True)
    u = (u - m) * lax.rsqrt(v + 1e-5) * g1_ref[...] + be1_ref[...]
    u = jnp.maximum(u, 0.0)
    y = jnp.dot(u, w2_ref[...], preferred_element_type=jnp.float32) + b2_ref[...]
    hn = hp_ref[...] + y
    hn_ref[...] = hn
    m2 = jnp.mean(hn, axis=-1, keepdims=True)
    v2 = jnp.mean((hn - m2) ** 2, axis=-1, keepdims=True)
    r_ref[...] = jnp.maximum(
        (hn - m2) * lax.rsqrt(v2 + 1e-5) * ng_ref[...] + nb_ref[...], 0.0)


def _layer_dense(aggr, hv, hp, cp, ng, nb):
    BN = 1280
    H2 = 2 * H
    row = lambda a, n: a.reshape(1, n)
    return pl.pallas_call(
        _layer_dense_body,
        grid=(NP // BN,),
        in_specs=[
            pl.BlockSpec((BN, H), lambda i: (i, 0)),
            pl.BlockSpec((BN, H), lambda i: (i, 0)),
            pl.BlockSpec((BN, H), lambda i: (i, 0)),
            pl.BlockSpec((H, H2), lambda i: (0, 0)),
            pl.BlockSpec((1, H2), lambda i: (0, 0)),
            pl.BlockSpec((1, H2), lambda i: (0, 0)),
            pl.BlockSpec((1, H2), lambda i: (0, 0)),
            pl.BlockSpec((H2, H), lambda i: (0, 0)),
            pl.BlockSpec((1, H), lambda i: (0, 0)),
            pl.BlockSpec((1, H), lambda i: (0, 0)),
            pl.BlockSpec((1, H), lambda i: (0, 0)),
        ],
        out_specs=[
            pl.BlockSpec((BN, H), lambda i: (i, 0)),
            pl.BlockSpec((BN, H), lambda i: (i, 0)),
        ],
        out_shape=[
            jax.ShapeDtypeStruct((NP, H), jnp.float32),
            jax.ShapeDtypeStruct((NP, H), jnp.float32),
        ],
    )(aggr, hv, hp, cp["W1"], row(cp["b1"], H2), row(cp["g1"], H2),
      row(cp["be1"], H2), cp["W2"], row(cp["b2"], H), row(ng, H), row(nb, H))


def _ab_body(hf_ref, w1a_ref, w1b_ref, b1_ref, a_ref, b_ref):
    hf = hf_ref[...]
    a_ref[...] = (jnp.dot(hf, w1a_ref[...],
                          preferred_element_type=jnp.float32) + b1_ref[...])
    b_ref[...] = jnp.dot(hf, w1b_ref[...], preferred_element_type=jnp.float32)


def _ab(hf, w1a, w1b, b1):
    BN = 1280
    return pl.pallas_call(
        _ab_body,
        grid=(NP // BN,),
        in_specs=[
            pl.BlockSpec((BN, H), lambda i: (i, 0)),
            pl.BlockSpec((H, H), lambda i: (0, 0)),
            pl.BlockSpec((H, H), lambda i: (0, 0)),
            pl.BlockSpec((1, H), lambda i: (0, 0)),
        ],
        out_specs=[
            pl.BlockSpec((BN, H), lambda i: (i, 0)),
            pl.BlockSpec((BN, H), lambda i: (i, 0)),
        ],
        out_shape=[
            jax.ShapeDtypeStruct((NP, H), jnp.float32),
            jax.ShapeDtypeStruct((NP, H), jnp.float32),
        ],
    )(hf, w1a, w1b, b1.reshape(1, H))


def kernel(x, edge_index, edge_attr, params):
    N = x.shape[0]
    E = edge_index.shape[1]
    layers = params["layers"]

    srcl, eidl, dstl, cnts = _build_lists(edge_index)

    x_p = jnp.pad(x, ((0, NP - N), (0, 0)))
    h0 = _enc_h(x_p, params["node_enc"]["W"], params["node_enc"]["b"])
    ea = _enc_ea(edge_attr, params["edge_enc"]["W"], params["edge_enc"]["b"])

    zeros_n = jnp.zeros((NP, H), jnp.float32)
    ea4 = ea.reshape(4 * E, Q)
    hv = h0
    hp = zeros_n
    for i in range(3):
        cp = layers[i]["conv"]
        nxt = layers[i + 1] if i < 2 else layers[0]
        aggr = _sc_layer(hv, ea4, srcl, eidl, dstl, cnts, cp["t"])
        hp, hv = _layer_dense(aggr, hv, hp, cp,
                              nxt["norm_g"], nxt["norm_b"])
    hf = hv

    m = params["mlp"]
    a_t, b_t = _ab(hf, m["W1"][:H, :], m["W1"][H:, :], m["b1"])

    EP = ((E + 128 * NW - 1) // (128 * NW)) * (128 * NW)
    ei_p = jnp.pad(edge_index, ((0, 0), (0, EP - E)))
    fp = jnp.zeros((4, H), jnp.float32)
    fp = fp.at[1].set(m["W2"][:, 0])
    fp = fp.at[2].set(m["W2"][:, 1])
    fp = fp.at[3, 0:2].set(m["b2"])
    out = _sc_final(a_t, b_t, ei_p, fp)
    return out[:E]
